# Initial kernel scaffold; baseline (speedup 1.0000x reference)
import jax
import jax.numpy as jnp
from jax.experimental import pallas as pl


def kernel(x, edge_index, batch, W1, b1, W_lin1, b_lin1, W_lin2, b_lin2):
    raise NotImplementedError("write your pallas kernel here")


# speedup vs baseline: 25.7277x; 25.7277x over previous
import functools

import jax
import jax.numpy as jnp
from jax import lax
from jax.experimental import pallas as pl
from jax.experimental.pallas import tpu as pltpu
from jax.experimental.pallas import tpu_sc as plsc

N = 50000
E = 800000
IN_DIM = 10
HID = 64
OUT = 2
G = 128

F = 16
N_PAD = 50176
E_PAD = 802816
CH = 128
N_TILES = 32
EDGES_T = E_PAD // N_TILES
NCHUNK = EDGES_T // CH
ROWS_T = N_PAD // 16
R_TC = 1024
NT_TC = N_PAD // R_TC

_mesh = plsc.VectorSubcoreMesh(core_axis_name="c", subcore_axis_name="s")


def _deg_body(dst_hbm, zeros_hbm, out_hbm, idx_v, ones_v, zbuf_v, deg_sh):
    c = lax.axis_index("c")
    s = lax.axis_index("s")
    wid = c * 16 + s
    rbase = s * ROWS_T
    pltpu.sync_copy(zeros_hbm.at[pl.ds(rbase, ROWS_T)], zbuf_v)
    pltpu.sync_copy(zbuf_v, deg_sh.at[pl.ds(rbase, ROWS_T)])
    for j in range(CH // 16):
        ones_v[pl.ds(j * 16, 16)] = jnp.full((16,), 1.0, jnp.float32)
    plsc.subcore_barrier()
    ebase = wid * EDGES_T

    def body(j, carry):
        pltpu.sync_copy(dst_hbm.at[pl.ds(ebase + j * CH, CH)], idx_v)
        pltpu.sync_copy(ones_v, deg_sh.at[idx_v], add=True)
        return carry

    lax.fori_loop(0, NCHUNK, body, 0)
    plsc.subcore_barrier()
    pltpu.sync_copy(deg_sh.at[pl.ds(rbase, ROWS_T)], zbuf_v)
    pltpu.sync_copy(zbuf_v, out_hbm.at[pl.ds(c * N_PAD + rbase, ROWS_T)])


_deg_kernel = functools.partial(
    pl.kernel,
    out_type=jax.ShapeDtypeStruct((2 * N_PAD,), jnp.float32),
    mesh=_mesh,
    scratch_types=[
        pltpu.VMEM((CH,), jnp.int32),
        pltpu.VMEM((CH,), jnp.float32),
        pltpu.VMEM((ROWS_T,), jnp.float32),
        pltpu.VMEM_SHARED((N_PAD,), jnp.float32),
    ],
)(_deg_body)


def _agg_body(src_hbm, dst_hbm, xs_hbm, zeros_hbm, out_hbm,
              idxs_v, idxd_v, rows_v, zbuf_v, acc_sh, sem):
    c = lax.axis_index("c")
    s = lax.axis_index("s")
    wid = c * 16 + s
    rbase = s * ROWS_T
    pltpu.sync_copy(zeros_hbm.at[pl.ds(rbase, ROWS_T)], zbuf_v)
    pltpu.sync_copy(zbuf_v, acc_sh.at[pl.ds(rbase, ROWS_T)])
    plsc.subcore_barrier()
    ebase = wid * EDGES_T

    def body(j, carry):
        pltpu.sync_copy(src_hbm.at[pl.ds(ebase + j * CH, CH)], idxs_v)
        pltpu.sync_copy(dst_hbm.at[pl.ds(ebase + j * CH, CH)], idxd_v)
        pltpu.async_copy(xs_hbm.at[idxs_v], rows_v, sem).wait()
        pltpu.sync_copy(rows_v, acc_sh.at[idxd_v], add=True)
        return carry

    lax.fori_loop(0, NCHUNK, body, 0)
    plsc.subcore_barrier()
    pltpu.sync_copy(acc_sh.at[pl.ds(rbase, ROWS_T)], zbuf_v)
    pltpu.sync_copy(zbuf_v, out_hbm.at[pl.ds(c * N_PAD + rbase, ROWS_T)])


_agg_kernel = functools.partial(
    pl.kernel,
    out_type=jax.ShapeDtypeStruct((2 * N_PAD, F), jnp.float32),
    mesh=_mesh,
    scratch_types=[
        pltpu.VMEM((CH,), jnp.int32),
        pltpu.VMEM((CH,), jnp.int32),
        pltpu.VMEM((CH, F), jnp.float32),
        pltpu.VMEM((ROWS_T, F), jnp.float32),
        pltpu.VMEM_SHARED((N_PAD, F), jnp.float32),
        pltpu.SemaphoreType.DMA,
    ],
    compiler_params=pltpu.CompilerParams(use_tc_tiling_on_sc=False),
)(_agg_body)


def _prep_body(d0_ref, d1_ref, x_ref, xs_ref, dis_ref):
    deg = d0_ref[...] + d1_ref[...] + 1.0
    dis = lax.rsqrt(deg)
    dis_ref[...] = dis
    xs_ref[...] = dis * x_ref[...]


def _prep_tc(d0, d1, x_pad):
    return pl.pallas_call(
        _prep_body,
        grid=(NT_TC,),
        in_specs=[
            pl.BlockSpec((R_TC, 1), lambda i: (i, 0)),
            pl.BlockSpec((R_TC, 1), lambda i: (i, 0)),
            pl.BlockSpec((R_TC, F), lambda i: (i, 0)),
        ],
        out_specs=[
            pl.BlockSpec((R_TC, F), lambda i: (i, 0)),
            pl.BlockSpec((R_TC, 1), lambda i: (i, 0)),
        ],
        out_shape=[
            jax.ShapeDtypeStruct((N_PAD, F), jnp.float32),
            jax.ShapeDtypeStruct((N_PAD, 1), jnp.float32),
        ],
    )(d0, d1, x_pad)


def _dense_body(a0_ref, a1_ref, xs_ref, dis_ref, bat_ref, w1_ref, b1_ref,
                wl1_ref, bl1_ref, wl2_ref, bl2_ref, out_ref, acc, cnt):
    i = pl.program_id(0)

    @pl.when(i == 0)
    def _():
        acc[...] = jnp.zeros_like(acc)
        cnt[...] = jnp.zeros_like(cnt)

    s = a0_ref[...] + a1_ref[...] + xs_ref[...]
    p = lax.dot_general(s, w1_ref[...], (((1,), (0,)), ((), ())),
                        preferred_element_type=jnp.float32)
    h = jnp.maximum(dis_ref[...] * p + b1_ref[...], 0.0)
    gids = lax.broadcasted_iota(jnp.int32, (1, G), 1)
    onehot = (bat_ref[...] == gids).astype(jnp.float32)
    acc[...] += lax.dot_general(onehot, h, (((0,), (0,)), ((), ())),
                                preferred_element_type=jnp.float32)
    cnt[...] += lax.dot_general(onehot, jnp.ones((R_TC, 1), jnp.float32),
                                (((0,), (0,)), ((), ())),
                                preferred_element_type=jnp.float32)

    @pl.when(i == NT_TC - 1)
    def _():
        pooled = acc[...] / jnp.maximum(cnt[...], 1.0)
        z1 = jnp.maximum(
            lax.dot_general(pooled, wl1_ref[...], (((1,), (0,)), ((), ())),
                            preferred_element_type=jnp.float32)
            + bl1_ref[...], 0.0)
        z2 = lax.dot_general(z1, wl2_ref[...], (((1,), (0,)), ((), ())),
                             preferred_element_type=jnp.float32) + bl2_ref[...]
        m = jnp.max(z2, axis=1, keepdims=True)
        lse = m + jnp.log(jnp.sum(jnp.exp(z2 - m), axis=1, keepdims=True))
        out_ref[...] = z2 - lse


def _dense_tc(a0, a1, xs, dis, bat, w1p, b1, wl1, bl1, wl2, bl2):
    whole = lambda shape: pl.BlockSpec(shape, lambda i: tuple(0 for _ in shape))
    return pl.pallas_call(
        _dense_body,
        grid=(NT_TC,),
        in_specs=[
            pl.BlockSpec((R_TC, F), lambda i: (i, 0)),
            pl.BlockSpec((R_TC, F), lambda i: (i, 0)),
            pl.BlockSpec((R_TC, F), lambda i: (i, 0)),
            pl.BlockSpec((R_TC, 1), lambda i: (i, 0)),
            pl.BlockSpec((R_TC, 1), lambda i: (i, 0)),
            whole((F, HID)),
            whole((1, HID)),
            whole((HID, HID)),
            whole((1, HID)),
            whole((HID, OUT)),
            whole((1, OUT)),
        ],
        out_specs=pl.BlockSpec((G, OUT), lambda i: (0, 0)),
        out_shape=jax.ShapeDtypeStruct((G, OUT), jnp.float32),
        scratch_shapes=[
            pltpu.VMEM((G, HID), jnp.float32),
            pltpu.VMEM((G, 1), jnp.float32),
        ],
    )(a0, a1, xs, dis, bat, w1p, b1, wl1, bl1, wl2, bl2)


def kernel(x, edge_index, batch, W1, b1, W_lin1, b_lin1, W_lin2, b_lin2):
    f32 = jnp.float32
    x_pad = jnp.zeros((N_PAD, F), f32).at[:N, :IN_DIM].set(x)
    pad_e = jnp.full((E_PAD - E,), N, jnp.int32)
    src = jnp.concatenate([edge_index[0], pad_e])
    dst = jnp.concatenate([edge_index[1], pad_e])
    bat = jnp.concatenate(
        [batch, jnp.full((N_PAD - N,), G, jnp.int32)]).reshape(N_PAD, 1)

    deg_p = _deg_kernel(dst, jnp.zeros((N_PAD,), f32))
    d0 = deg_p[:N_PAD].reshape(N_PAD, 1)
    d1 = deg_p[N_PAD:].reshape(N_PAD, 1)
    xs, dis = _prep_tc(d0, d1, x_pad)

    asum_p = _agg_kernel(src, dst, xs, jnp.zeros((N_PAD, F), f32))

    w1p = jnp.zeros((F, HID), f32).at[:IN_DIM].set(W1)
    out = _dense_tc(asum_p[:N_PAD], asum_p[N_PAD:], xs, dis, bat,
                    w1p, b1.reshape(1, HID),
                    W_lin1, b_lin1.reshape(1, HID),
                    W_lin2, b_lin2.reshape(1, OUT))
    return out


# speedup vs baseline: 48.2987x; 1.8773x over previous
import functools

import jax
import jax.numpy as jnp
from jax import lax
from jax.experimental import pallas as pl
from jax.experimental.pallas import tpu as pltpu
from jax.experimental.pallas import tpu_sc as plsc

N = 50000
E = 800000
IN_DIM = 10
HID = 64
OUT = 2
G = 128

F = 16
N_PAD = 50176
E_PAD = 802816
CH = 128
N_TILES = 32
EDGES_T = E_PAD // N_TILES
NCHUNK = EDGES_T // CH
ROWS_T = N_PAD // 16
R_TC = 1024
NT_TC = N_PAD // R_TC

_mesh = plsc.VectorSubcoreMesh(core_axis_name="c", subcore_axis_name="s")


K_DEG = 14
K_AGG = 7


def _deg_body(dst_hbm, zeros_hbm, out_hbm, idxb_v, ones_v, zbuf_v, deg_sh, sem):
    c = lax.axis_index("c")
    s = lax.axis_index("s")
    wid = c * 16 + s
    rbase = s * ROWS_T
    pltpu.sync_copy(zeros_hbm.at[pl.ds(rbase, ROWS_T)], zbuf_v)
    pltpu.sync_copy(zbuf_v, deg_sh.at[pl.ds(rbase, ROWS_T)])
    pltpu.sync_copy(dst_hbm.at[pl.ds(wid * NCHUNK, NCHUNK)], idxb_v)
    for j in range(CH // 16):
        ones_v[pl.ds(j * 16, 16)] = jnp.full((16,), 1.0, jnp.float32)
    plsc.subcore_barrier()

    def body(g, carry):
        hs = [pltpu.async_copy(ones_v, deg_sh.at[idxb_v.at[g * K_DEG + b]],
                               sem, add=True)
              for b in range(K_DEG)]
        for h in hs:
            h.wait()
        return carry

    lax.fori_loop(0, NCHUNK // K_DEG, body, 0)
    plsc.subcore_barrier()
    pltpu.sync_copy(deg_sh.at[pl.ds(rbase, ROWS_T)], zbuf_v)
    pltpu.sync_copy(zbuf_v, out_hbm.at[pl.ds(c * N_PAD + rbase, ROWS_T)])


_deg_kernel = functools.partial(
    pl.kernel,
    out_type=jax.ShapeDtypeStruct((2 * N_PAD,), jnp.float32),
    mesh=_mesh,
    scratch_types=[
        pltpu.VMEM((NCHUNK, CH), jnp.int32),
        pltpu.VMEM((CH,), jnp.float32),
        pltpu.VMEM((ROWS_T,), jnp.float32),
        pltpu.VMEM_SHARED((N_PAD,), jnp.float32),
        pltpu.SemaphoreType.DMA,
    ],
    compiler_params=pltpu.CompilerParams(use_tc_tiling_on_sc=False),
)(_deg_body)


def _agg_body(src_hbm, dst_hbm, xs_hbm, zeros_hbm, out_hbm,
              srcb_v, dstb_v, rows_v, zbuf_v, acc_sh, gsem, ssem):
    c = lax.axis_index("c")
    s = lax.axis_index("s")
    wid = c * 16 + s
    rbase = s * ROWS_T
    for t in range(4):
        pltpu.sync_copy(zeros_hbm.at[pl.ds(rbase + t * (ROWS_T // 4),
                                           ROWS_T // 4)], zbuf_v)
        pltpu.sync_copy(zbuf_v, acc_sh.at[pl.ds(rbase + t * (ROWS_T // 4),
                                                ROWS_T // 4)])
    pltpu.sync_copy(src_hbm.at[pl.ds(wid * NCHUNK, NCHUNK)], srcb_v)
    pltpu.sync_copy(dst_hbm.at[pl.ds(wid * NCHUNK, NCHUNK)], dstb_v)
    plsc.subcore_barrier()

    def body(g, carry):
        gh = [pltpu.async_copy(xs_hbm.at[srcb_v.at[g * K_AGG + b]],
                               rows_v.at[b], gsem)
              for b in range(K_AGG)]
        for h in gh:
            h.wait()
        sh = [pltpu.async_copy(rows_v.at[b],
                               acc_sh.at[dstb_v.at[g * K_AGG + b]],
                               ssem, add=True)
              for b in range(K_AGG)]
        for h in sh:
            h.wait()
        return carry

    lax.fori_loop(0, NCHUNK // K_AGG, body, 0)
    plsc.subcore_barrier()
    for t in range(4):
        pltpu.sync_copy(acc_sh.at[pl.ds(rbase + t * (ROWS_T // 4),
                                        ROWS_T // 4)], zbuf_v)
        pltpu.sync_copy(zbuf_v, out_hbm.at[pl.ds(c * N_PAD + rbase
                                                 + t * (ROWS_T // 4),
                                                 ROWS_T // 4)])


_agg_kernel = functools.partial(
    pl.kernel,
    out_type=jax.ShapeDtypeStruct((2 * N_PAD, F), jnp.float32),
    mesh=_mesh,
    scratch_types=[
        pltpu.VMEM((NCHUNK, CH), jnp.int32),
        pltpu.VMEM((NCHUNK, CH), jnp.int32),
        pltpu.VMEM((K_AGG, CH, F), jnp.float32),
        pltpu.VMEM((ROWS_T // 4, F), jnp.float32),
        pltpu.VMEM_SHARED((N_PAD, F), jnp.float32),
        pltpu.SemaphoreType.DMA,
        pltpu.SemaphoreType.DMA,
    ],
    compiler_params=pltpu.CompilerParams(use_tc_tiling_on_sc=False),
)(_agg_body)


def _prep_body(d0_ref, d1_ref, x_ref, xs_ref, dis_ref):
    deg = d0_ref[...] + d1_ref[...] + 1.0
    dis = lax.rsqrt(deg)
    dis_ref[...] = dis
    xs_ref[...] = dis * x_ref[...]


def _prep_tc(d0, d1, x_pad):
    return pl.pallas_call(
        _prep_body,
        grid=(NT_TC,),
        in_specs=[
            pl.BlockSpec((R_TC, 1), lambda i: (i, 0)),
            pl.BlockSpec((R_TC, 1), lambda i: (i, 0)),
            pl.BlockSpec((R_TC, F), lambda i: (i, 0)),
        ],
        out_specs=[
            pl.BlockSpec((R_TC, F), lambda i: (i, 0)),
            pl.BlockSpec((R_TC, 1), lambda i: (i, 0)),
        ],
        out_shape=[
            jax.ShapeDtypeStruct((N_PAD, F), jnp.float32),
            jax.ShapeDtypeStruct((N_PAD, 1), jnp.float32),
        ],
    )(d0, d1, x_pad)


def _dense_body(a0_ref, a1_ref, xs_ref, dis_ref, bat_ref, w1_ref, b1_ref,
                wl1_ref, bl1_ref, wl2_ref, bl2_ref, out_ref, acc, cnt):
    i = pl.program_id(0)

    @pl.when(i == 0)
    def _():
        acc[...] = jnp.zeros_like(acc)
        cnt[...] = jnp.zeros_like(cnt)

    s = a0_ref[...] + a1_ref[...] + xs_ref[...]
    p = lax.dot_general(s, w1_ref[...], (((1,), (0,)), ((), ())),
                        preferred_element_type=jnp.float32)
    h = jnp.maximum(dis_ref[...] * p + b1_ref[...], 0.0)
    gids = lax.broadcasted_iota(jnp.int32, (1, G), 1)
    onehot = (bat_ref[...] == gids).astype(jnp.float32)
    acc[...] += lax.dot_general(onehot, h, (((0,), (0,)), ((), ())),
                                preferred_element_type=jnp.float32)
    cnt[...] += lax.dot_general(onehot, jnp.ones((R_TC, 1), jnp.float32),
                                (((0,), (0,)), ((), ())),
                                preferred_element_type=jnp.float32)

    @pl.when(i == NT_TC - 1)
    def _():
        pooled = acc[...] / jnp.maximum(cnt[...], 1.0)
        z1 = jnp.maximum(
            lax.dot_general(pooled, wl1_ref[...], (((1,), (0,)), ((), ())),
                            preferred_element_type=jnp.float32)
            + bl1_ref[...], 0.0)
        z2 = lax.dot_general(z1, wl2_ref[...], (((1,), (0,)), ((), ())),
                             preferred_element_type=jnp.float32) + bl2_ref[...]
        m = jnp.max(z2, axis=1, keepdims=True)
        lse = m + jnp.log(jnp.sum(jnp.exp(z2 - m), axis=1, keepdims=True))
        out_ref[...] = z2 - lse


def _dense_tc(a0, a1, xs, dis, bat, w1p, b1, wl1, bl1, wl2, bl2):
    whole = lambda shape: pl.BlockSpec(shape, lambda i: tuple(0 for _ in shape))
    return pl.pallas_call(
        _dense_body,
        grid=(NT_TC,),
        in_specs=[
            pl.BlockSpec((R_TC, F), lambda i: (i, 0)),
            pl.BlockSpec((R_TC, F), lambda i: (i, 0)),
            pl.BlockSpec((R_TC, F), lambda i: (i, 0)),
            pl.BlockSpec((R_TC, 1), lambda i: (i, 0)),
            pl.BlockSpec((R_TC, 1), lambda i: (i, 0)),
            whole((F, HID)),
            whole((1, HID)),
            whole((HID, HID)),
            whole((1, HID)),
            whole((HID, OUT)),
            whole((1, OUT)),
        ],
        out_specs=pl.BlockSpec((G, OUT), lambda i: (0, 0)),
        out_shape=jax.ShapeDtypeStruct((G, OUT), jnp.float32),
        scratch_shapes=[
            pltpu.VMEM((G, HID), jnp.float32),
            pltpu.VMEM((G, 1), jnp.float32),
        ],
    )(a0, a1, xs, dis, bat, w1p, b1, wl1, bl1, wl2, bl2)


def kernel(x, edge_index, batch, W1, b1, W_lin1, b_lin1, W_lin2, b_lin2):
    f32 = jnp.float32
    x_pad = jnp.zeros((N_PAD, F), f32).at[:N, :IN_DIM].set(x)
    pad_e = jnp.full((E_PAD - E,), N, jnp.int32)
    src = jnp.concatenate([edge_index[0], pad_e]).reshape(E_PAD // CH, CH)
    dst = jnp.concatenate([edge_index[1], pad_e]).reshape(E_PAD // CH, CH)
    bat = jnp.concatenate(
        [batch, jnp.full((N_PAD - N,), G, jnp.int32)]).reshape(N_PAD, 1)

    deg_p = _deg_kernel(dst, jnp.zeros((N_PAD,), f32))
    d0 = deg_p[:N_PAD].reshape(N_PAD, 1)
    d1 = deg_p[N_PAD:].reshape(N_PAD, 1)
    xs, dis = _prep_tc(d0, d1, x_pad)

    asum_p = _agg_kernel(src, dst, xs, jnp.zeros((N_PAD, F), f32))

    w1p = jnp.zeros((F, HID), f32).at[:IN_DIM].set(W1)
    out = _dense_tc(asum_p[:N_PAD], asum_p[N_PAD:], xs, dis, bat,
                    w1p, b1.reshape(1, HID),
                    W_lin1, b_lin1.reshape(1, HID),
                    W_lin2, b_lin2.reshape(1, OUT))
    return out


# speedup vs baseline: 67.0242x; 1.3877x over previous
import functools

import jax
import jax.numpy as jnp
from jax import lax
from jax.experimental import pallas as pl
from jax.experimental.pallas import tpu as pltpu
from jax.experimental.pallas import tpu_sc as plsc

N = 50000
E = 800000
IN_DIM = 10
HID = 64
OUT = 2
G = 128

F = 16
CH = 128
NCHE = E // CH
N_TILES = 32
CPT = NCHE // N_TILES
XTRA = NCHE - CPT * N_TILES
K_DEG = 13
G_DEG = CPT // K_DEG
K_AGG = 5
G_AGG = CPT // K_AGG

N2 = 51200
ROWS_T = N2 // 16
ZR = 800

R_TC = 2048
NT_TC = N2 // R_TC

_mesh = plsc.VectorSubcoreMesh(core_axis_name="c", subcore_axis_name="s")


def _deg_body(dst_hbm, zeros_hbm, out_hbm, idxb_v, ones_v, zbuf_v, deg_sh,
              sem):
    c = lax.axis_index("c")
    s = lax.axis_index("s")
    wid = c * 16 + s
    rbase = s * ROWS_T

    for t in range(ROWS_T // ZR):
        pltpu.sync_copy(zeros_hbm.at[pl.ds(rbase + t * ZR, ZR)], zbuf_v)
        pltpu.sync_copy(zbuf_v, deg_sh.at[pl.ds(rbase + t * ZR, ZR)])
    pltpu.sync_copy(dst_hbm.at[pl.ds(wid * CPT, CPT)], idxb_v)
    for j in range(CH // 16):
        ones_v[pl.ds(j * 16, 16)] = jnp.full((16,), 1.0, jnp.float32)
    plsc.subcore_barrier()

    def body(g, carry):
        hs = [pltpu.async_copy(ones_v, deg_sh.at[idxb_v.at[g * K_DEG + b]],
                               sem, add=True)
              for b in range(K_DEG)]
        for h in hs:
            h.wait()
        return carry

    lax.fori_loop(0, G_DEG, body, 0)

    @pl.when(wid < XTRA)
    def _():
        pltpu.sync_copy(dst_hbm.at[N_TILES * CPT + wid], idxb_v.at[0])
        pltpu.async_copy(ones_v, deg_sh.at[idxb_v.at[0]], sem,
                         add=True).wait()

    plsc.subcore_barrier()
    for t in range(ROWS_T // ZR):
        pltpu.sync_copy(deg_sh.at[pl.ds(rbase + t * ZR, ZR)], zbuf_v)
        pltpu.sync_copy(zbuf_v, out_hbm.at[pl.ds(c * N2 + rbase + t * ZR, ZR)])


_deg_kernel = functools.partial(
    pl.kernel,
    out_type=jax.ShapeDtypeStruct((2 * N2,), jnp.float32),
    mesh=_mesh,
    scratch_types=[
        pltpu.VMEM((CPT, CH), jnp.int32),
        pltpu.VMEM((CH,), jnp.float32),
        pltpu.VMEM((ZR,), jnp.float32),
        pltpu.VMEM_SHARED((N2,), jnp.float32),
        pltpu.SemaphoreType.DMA,
    ],
    compiler_params=pltpu.CompilerParams(use_tc_tiling_on_sc=False),
)(_deg_body)


def _agg_body(src_hbm, dst_hbm, xs_hbm, zeros_hbm, out_hbm,
              srcb_v, dstb_v, rows_v, zbuf_v, acc_sh, gsem, ssem):
    c = lax.axis_index("c")
    s = lax.axis_index("s")
    wid = c * 16 + s
    rbase = s * ROWS_T

    for t in range(ROWS_T // ZR):
        pltpu.sync_copy(zeros_hbm.at[pl.ds(rbase + t * ZR, ZR)], zbuf_v)
        pltpu.sync_copy(zbuf_v, acc_sh.at[pl.ds(rbase + t * ZR, ZR)])
    pltpu.sync_copy(src_hbm.at[pl.ds(wid * CPT, CPT)], srcb_v)
    pltpu.sync_copy(dst_hbm.at[pl.ds(wid * CPT, CPT)], dstb_v)
    plsc.subcore_barrier()

    def body(g, carry):
        gh = [pltpu.async_copy(xs_hbm.at[srcb_v.at[g * K_AGG + b]],
                               rows_v.at[b], gsem)
              for b in range(K_AGG)]
        for h in gh:
            h.wait()
        sh = [pltpu.async_copy(rows_v.at[b],
                               acc_sh.at[dstb_v.at[g * K_AGG + b]],
                               ssem, add=True)
              for b in range(K_AGG)]
        for h in sh:
            h.wait()
        return carry

    lax.fori_loop(0, G_AGG, body, 0)

    @pl.when(wid < XTRA)
    def _():
        pltpu.sync_copy(src_hbm.at[N_TILES * CPT + wid], srcb_v.at[0])
        pltpu.sync_copy(dst_hbm.at[N_TILES * CPT + wid], dstb_v.at[0])
        pltpu.async_copy(xs_hbm.at[srcb_v.at[0]], rows_v.at[0], gsem).wait()
        pltpu.async_copy(rows_v.at[0], acc_sh.at[dstb_v.at[0]], ssem,
                         add=True).wait()

    plsc.subcore_barrier()
    for t in range(ROWS_T // ZR):
        pltpu.sync_copy(acc_sh.at[pl.ds(rbase + t * ZR, ZR)], zbuf_v)
        pltpu.sync_copy(zbuf_v, out_hbm.at[pl.ds(c * N2 + rbase + t * ZR, ZR)])


_agg_kernel = functools.partial(
    pl.kernel,
    out_type=jax.ShapeDtypeStruct((2 * N2, F), jnp.float32),
    mesh=_mesh,
    scratch_types=[
        pltpu.VMEM((CPT, CH), jnp.int32),
        pltpu.VMEM((CPT, CH), jnp.int32),
        pltpu.VMEM((K_AGG, CH, F), jnp.float32),
        pltpu.VMEM((ZR, F), jnp.float32),
        pltpu.VMEM_SHARED((N2, F), jnp.float32),
        pltpu.SemaphoreType.DMA,
        pltpu.SemaphoreType.DMA,
    ],
    compiler_params=pltpu.CompilerParams(use_tc_tiling_on_sc=False),
)(_agg_body)


def _prep_body(d0_ref, d1_ref, x_ref, xs_ref):
    dis = lax.rsqrt(d0_ref[...] + d1_ref[...] + 1.0)
    dis2 = dis.reshape(R_TC, 1)
    xs_ref[...] = jnp.concatenate(
        [dis2 * x_ref[...], jnp.zeros((R_TC, F - IN_DIM), jnp.float32)],
        axis=1)


def _prep_tc(deg_p, x_pad):
    return pl.pallas_call(
        _prep_body,
        grid=(NT_TC,),
        in_specs=[
            pl.BlockSpec((R_TC,), lambda i: (i,)),
            pl.BlockSpec((R_TC,), lambda i: (i + NT_TC,)),
            pl.BlockSpec((R_TC, IN_DIM), lambda i: (i, 0)),
        ],
        out_specs=pl.BlockSpec((R_TC, F), lambda i: (i, 0)),
        out_shape=jax.ShapeDtypeStruct((N2, F), jnp.float32),
    )(deg_p, deg_p, x_pad)


def _dense_body(a0_ref, a1_ref, xs_ref, d0_ref, d1_ref, bat_ref, w1_ref,
                b1_ref, wl1_ref, bl1_ref, wl2_ref, bl2_ref, out_ref,
                acc, cnt):
    i = pl.program_id(0)

    @pl.when(i == 0)
    def _():
        acc[...] = jnp.zeros_like(acc)
        cnt[...] = jnp.zeros_like(cnt)

    dis = lax.rsqrt(d0_ref[...] + d1_ref[...] + 1.0).reshape(R_TC, 1)
    s = a0_ref[...] + a1_ref[...] + xs_ref[...]
    p = lax.dot_general(s, w1_ref[...], (((1,), (0,)), ((), ())),
                        preferred_element_type=jnp.float32)
    h = jnp.maximum(dis * p + b1_ref[...], 0.0)
    gids = lax.broadcasted_iota(jnp.int32, (1, G), 1)
    onehot = (bat_ref[...].reshape(R_TC, 1) == gids).astype(jnp.float32)
    acc[...] += lax.dot_general(onehot, h, (((0,), (0,)), ((), ())),
                                preferred_element_type=jnp.float32)
    cnt[...] += lax.dot_general(onehot, jnp.ones((R_TC, 1), jnp.float32),
                                (((0,), (0,)), ((), ())),
                                preferred_element_type=jnp.float32)

    @pl.when(i == NT_TC - 1)
    def _():
        pooled = acc[...] / jnp.maximum(cnt[...], 1.0)
        z1 = jnp.maximum(
            lax.dot_general(pooled, wl1_ref[...], (((1,), (0,)), ((), ())),
                            preferred_element_type=jnp.float32)
            + bl1_ref[...], 0.0)
        z2 = lax.dot_general(z1, wl2_ref[...], (((1,), (0,)), ((), ())),
                             preferred_element_type=jnp.float32) + bl2_ref[...]
        m = jnp.max(z2, axis=1, keepdims=True)
        lse = m + jnp.log(jnp.sum(jnp.exp(z2 - m), axis=1, keepdims=True))
        out_ref[...] = z2 - lse


def _dense_tc(asum_p, xs, deg_p, bat_pad, w1p, b1, wl1, bl1, wl2, bl2):
    whole = lambda shape: pl.BlockSpec(shape, lambda i: tuple(0 for _ in shape))
    return pl.pallas_call(
        _dense_body,
        grid=(NT_TC,),
        in_specs=[
            pl.BlockSpec((R_TC, F), lambda i: (i, 0)),
            pl.BlockSpec((R_TC, F), lambda i: (i + NT_TC, 0)),
            pl.BlockSpec((R_TC, F), lambda i: (i, 0)),
            pl.BlockSpec((R_TC,), lambda i: (i,)),
            pl.BlockSpec((R_TC,), lambda i: (i + NT_TC,)),
            pl.BlockSpec((R_TC,), lambda i: (i,)),
            whole((F, HID)),
            whole((1, HID)),
            whole((HID, HID)),
            whole((1, HID)),
            whole((HID, OUT)),
            whole((1, OUT)),
        ],
        out_specs=pl.BlockSpec((G, OUT), lambda i: (0, 0)),
        out_shape=jax.ShapeDtypeStruct((G, OUT), jnp.float32),
        scratch_shapes=[
            pltpu.VMEM((G, HID), jnp.float32),
            pltpu.VMEM((G, 1), jnp.float32),
        ],
    )(asum_p, asum_p, xs, deg_p, deg_p, bat_pad, w1p, b1, wl1, bl1, wl2, bl2)


def kernel(x, edge_index, batch, W1, b1, W_lin1, b_lin1, W_lin2, b_lin2):
    f32 = jnp.float32
    src = edge_index[0].reshape(NCHE, CH)
    dst = edge_index[1].reshape(NCHE, CH)
    x_pad = jnp.pad(x, ((0, N2 - N), (0, 0)))
    bat_pad = jnp.pad(batch, (0, N2 - N), constant_values=G)

    deg_p = _deg_kernel(dst, jnp.zeros((N2,), f32))
    xs = _prep_tc(deg_p, x_pad)
    asum_p = _agg_kernel(src, dst, xs, jnp.zeros((N2, F), f32))

    w1p = jnp.zeros((F, HID), f32).at[:IN_DIM].set(W1)
    return _dense_tc(asum_p, xs, deg_p, bat_pad,
                     w1p, b1.reshape(1, HID),
                     W_lin1, b_lin1.reshape(1, HID),
                     W_lin2, b_lin2.reshape(1, OUT))


# speedup vs baseline: 75.5287x; 1.1269x over previous
import functools

import jax
import jax.numpy as jnp
from jax import lax
from jax.experimental import pallas as pl
from jax.experimental.pallas import tpu as pltpu
from jax.experimental.pallas import tpu_sc as plsc

N = 50000
E = 800000
IN_DIM = 10
HID = 64
OUT = 2
G = 128

F = 16
CH = 128
NCHE = E // CH
N_TILES = 32
CPT = NCHE // N_TILES
XTRA = NCHE - CPT * N_TILES
K_DEG = 13
G_DEG = CPT // K_DEG
K_AGG = 5
G_AGG = CPT // K_AGG

N2 = 51200
ROWS_T = N2 // 16
ZR = 800

R_TC = 2048
NT_TC = N2 // R_TC

_mesh = plsc.VectorSubcoreMesh(core_axis_name="c", subcore_axis_name="s")


def _stage_chunk(flat_v, stg_v, b, pos):
    for j in range(CH // 16):
        stg_v[b, pl.ds(j * 16, 16)] = flat_v[pl.ds(pos * CH + j * 16, 16)]


def _deg_body(ei_hbm, zeros_hbm, out_hbm, idxb_v, stg_v, ones_v, zbuf_v,
              deg_sh, sem):
    c = lax.axis_index("c")
    s = lax.axis_index("s")
    wid = c * 16 + s
    rbase = s * ROWS_T

    for t in range(ROWS_T // ZR):
        pltpu.sync_copy(zeros_hbm.at[pl.ds(rbase + t * ZR, ZR)], zbuf_v)
        pltpu.sync_copy(zbuf_v, deg_sh.at[pl.ds(rbase + t * ZR, ZR)])
    pltpu.sync_copy(ei_hbm.at[1, pl.ds(wid * CPT * CH, CPT * CH)], idxb_v)
    for j in range(CH // 16):
        ones_v[pl.ds(j * 16, 16)] = jnp.full((16,), 1.0, jnp.float32)
    plsc.subcore_barrier()

    def body(g, carry):
        for b in range(K_DEG):
            _stage_chunk(idxb_v, stg_v, b, g * K_DEG + b)
        hs = [pltpu.async_copy(ones_v, deg_sh.at[stg_v.at[b]], sem, add=True)
              for b in range(K_DEG)]
        for h in hs:
            h.wait()
        return carry

    lax.fori_loop(0, G_DEG, body, 0)

    @pl.when(wid < XTRA)
    def _():
        pltpu.sync_copy(ei_hbm.at[1, pl.ds((N_TILES * CPT + wid) * CH, CH)],
                        stg_v.at[0])
        pltpu.async_copy(ones_v, deg_sh.at[stg_v.at[0]], sem,
                         add=True).wait()

    plsc.subcore_barrier()
    for t in range(ROWS_T // ZR):
        pltpu.sync_copy(deg_sh.at[pl.ds(rbase + t * ZR, ZR)], zbuf_v)
        pltpu.sync_copy(zbuf_v, out_hbm.at[pl.ds(c * N2 + rbase + t * ZR, ZR)])


_deg_kernel = functools.partial(
    pl.kernel,
    out_type=jax.ShapeDtypeStruct((2 * N2,), jnp.float32),
    mesh=_mesh,
    scratch_types=[
        pltpu.VMEM((CPT * CH,), jnp.int32),
        pltpu.VMEM((K_DEG, CH), jnp.int32),
        pltpu.VMEM((CH,), jnp.float32),
        pltpu.VMEM((ZR,), jnp.float32),
        pltpu.VMEM_SHARED((N2,), jnp.float32),
        pltpu.SemaphoreType.DMA,
    ],
    compiler_params=pltpu.CompilerParams(use_tc_tiling_on_sc=False),
)(_deg_body)


def _agg_body(ei_hbm, xs_hbm, zeros_hbm, out_hbm,
              srcb_v, dstb_v, stg_v, rows_v, zbuf_v, acc_sh, gsem, ssem):
    c = lax.axis_index("c")
    s = lax.axis_index("s")
    wid = c * 16 + s
    rbase = s * ROWS_T

    for t in range(ROWS_T // ZR):
        pltpu.sync_copy(zeros_hbm.at[pl.ds(rbase + t * ZR, ZR)], zbuf_v)
        pltpu.sync_copy(zbuf_v, acc_sh.at[pl.ds(rbase + t * ZR, ZR)])
    pltpu.sync_copy(ei_hbm.at[0, pl.ds(wid * CPT * CH, CPT * CH)], srcb_v)
    pltpu.sync_copy(ei_hbm.at[1, pl.ds(wid * CPT * CH, CPT * CH)], dstb_v)
    plsc.subcore_barrier()

    def body(g, carry):
        gh = [pltpu.async_copy(
                  xs_hbm.at[srcb_v.at[pl.ds((g * K_AGG + b) * CH, CH)]],
                  rows_v.at[b], gsem)
              for b in range(K_AGG)]
        for b in range(K_AGG):
            _stage_chunk(dstb_v, stg_v, b, g * K_AGG + b)
        for h in gh:
            h.wait()
        sh = [pltpu.async_copy(rows_v.at[b], acc_sh.at[stg_v.at[b]],
                               ssem, add=True)
              for b in range(K_AGG)]
        for h in sh:
            h.wait()
        return carry

    lax.fori_loop(0, G_AGG, body, 0)

    @pl.when(wid < XTRA)
    def _():
        pltpu.sync_copy(ei_hbm.at[0, pl.ds((N_TILES * CPT + wid) * CH, CH)],
                        srcb_v.at[pl.ds(0, CH)])
        pltpu.sync_copy(ei_hbm.at[1, pl.ds((N_TILES * CPT + wid) * CH, CH)],
                        stg_v.at[0])
        pltpu.async_copy(xs_hbm.at[srcb_v.at[pl.ds(0, CH)]], rows_v.at[0],
                         gsem).wait()
        pltpu.async_copy(rows_v.at[0], acc_sh.at[stg_v.at[0]], ssem,
                         add=True).wait()

    plsc.subcore_barrier()
    for t in range(ROWS_T // ZR):
        pltpu.sync_copy(acc_sh.at[pl.ds(rbase + t * ZR, ZR)], zbuf_v)
        pltpu.sync_copy(zbuf_v, out_hbm.at[pl.ds(c * N2 + rbase + t * ZR, ZR)])


_agg_kernel = functools.partial(
    pl.kernel,
    out_type=jax.ShapeDtypeStruct((2 * N2, F), jnp.float32),
    mesh=_mesh,
    scratch_types=[
        pltpu.VMEM((CPT * CH,), jnp.int32),
        pltpu.VMEM((CPT * CH,), jnp.int32),
        pltpu.VMEM((K_AGG, CH), jnp.int32),
        pltpu.VMEM((K_AGG, CH, F), jnp.float32),
        pltpu.VMEM((ZR, F), jnp.float32),
        pltpu.VMEM_SHARED((N2, F), jnp.float32),
        pltpu.SemaphoreType.DMA,
        pltpu.SemaphoreType.DMA,
    ],
    compiler_params=pltpu.CompilerParams(use_tc_tiling_on_sc=False),
)(_agg_body)


def _prep_body(d0_ref, d1_ref, x_ref, xs_ref):
    i = pl.program_id(0)
    dis = lax.rsqrt(d0_ref[...] + d1_ref[...] + 1.0)
    dis2 = dis.reshape(R_TC, 1)
    nid = i * R_TC + lax.broadcasted_iota(jnp.int32, (R_TC, 1), 0)
    val = jnp.where(nid < N, dis2 * x_ref[...], 0.0)
    xs_ref[...] = jnp.concatenate(
        [val, jnp.zeros((R_TC, F - IN_DIM), jnp.float32)], axis=1)


def _prep_tc(deg_p, x):
    return pl.pallas_call(
        _prep_body,
        grid=(NT_TC,),
        in_specs=[
            pl.BlockSpec((R_TC,), lambda i: (i,)),
            pl.BlockSpec((R_TC,), lambda i: (i + NT_TC,)),
            pl.BlockSpec((R_TC, IN_DIM), lambda i: (i, 0)),
        ],
        out_specs=pl.BlockSpec((R_TC, F), lambda i: (i, 0)),
        out_shape=jax.ShapeDtypeStruct((N2, F), jnp.float32),
    )(deg_p, deg_p, x)


def _dense_body(a0_ref, a1_ref, xs_ref, d0_ref, d1_ref, bat_ref, w1_ref,
                b1_ref, wl1_ref, bl1_ref, wl2_ref, bl2_ref, out_ref,
                acc, cnt):
    i = pl.program_id(0)

    @pl.when(i == 0)
    def _():
        acc[...] = jnp.zeros_like(acc)
        cnt[...] = jnp.zeros_like(cnt)

    dis = lax.rsqrt(d0_ref[...] + d1_ref[...] + 1.0).reshape(R_TC, 1)
    s = a0_ref[...] + a1_ref[...] + xs_ref[...]
    p = lax.dot_general(s, w1_ref[...], (((1,), (0,)), ((), ())),
                        preferred_element_type=jnp.float32)
    h = jnp.maximum(dis * p + b1_ref[...], 0.0)
    gids = lax.broadcasted_iota(jnp.int32, (1, G), 1)
    nid = i * R_TC + lax.broadcasted_iota(jnp.int32, (R_TC, 1), 0)
    onehot = ((bat_ref[...].reshape(R_TC, 1) == gids)
              & (nid < N)).astype(jnp.float32)
    acc[...] += lax.dot_general(onehot, h, (((0,), (0,)), ((), ())),
                                preferred_element_type=jnp.float32)
    cnt[...] += lax.dot_general(onehot, jnp.ones((R_TC, 1), jnp.float32),
                                (((0,), (0,)), ((), ())),
                                preferred_element_type=jnp.float32)

    @pl.when(i == NT_TC - 1)
    def _():
        pooled = acc[...] / jnp.maximum(cnt[...], 1.0)
        z1 = jnp.maximum(
            lax.dot_general(pooled, wl1_ref[...], (((1,), (0,)), ((), ())),
                            preferred_element_type=jnp.float32)
            + bl1_ref[...], 0.0)
        z2 = lax.dot_general(z1, wl2_ref[...], (((1,), (0,)), ((), ())),
                             preferred_element_type=jnp.float32) + bl2_ref[...]
        m = jnp.max(z2, axis=1, keepdims=True)
        lse = m + jnp.log(jnp.sum(jnp.exp(z2 - m), axis=1, keepdims=True))
        out_ref[...] = z2 - lse


def _dense_tc(asum_p, xs, deg_p, bat, w1p, b1, wl1, bl1, wl2, bl2):
    whole = lambda shape: pl.BlockSpec(shape, lambda i: tuple(0 for _ in shape))
    return pl.pallas_call(
        _dense_body,
        grid=(NT_TC,),
        in_specs=[
            pl.BlockSpec((R_TC, F), lambda i: (i, 0)),
            pl.BlockSpec((R_TC, F), lambda i: (i + NT_TC, 0)),
            pl.BlockSpec((R_TC, F), lambda i: (i, 0)),
            pl.BlockSpec((R_TC,), lambda i: (i,)),
            pl.BlockSpec((R_TC,), lambda i: (i + NT_TC,)),
            pl.BlockSpec((R_TC,), lambda i: (i,)),
            whole((F, HID)),
            whole((1, HID)),
            whole((HID, HID)),
            whole((1, HID)),
            whole((HID, OUT)),
            whole((1, OUT)),
        ],
        out_specs=pl.BlockSpec((G, OUT), lambda i: (0, 0)),
        out_shape=jax.ShapeDtypeStruct((G, OUT), jnp.float32),
        scratch_shapes=[
            pltpu.VMEM((G, HID), jnp.float32),
            pltpu.VMEM((G, 1), jnp.float32),
        ],
    )(asum_p, asum_p, xs, deg_p, deg_p, bat, w1p, b1, wl1, bl1, wl2, bl2)


def kernel(x, edge_index, batch, W1, b1, W_lin1, b_lin1, W_lin2, b_lin2):
    f32 = jnp.float32
    deg_p = _deg_kernel(edge_index, jnp.zeros((N2,), f32))
    xs = _prep_tc(deg_p, x)
    asum_p = _agg_kernel(edge_index, xs, jnp.zeros((N2, F), f32))

    w1p = jnp.zeros((F, HID), f32).at[:IN_DIM].set(W1)
    return _dense_tc(asum_p, xs, deg_p, batch,
                     w1p, b1.reshape(1, HID),
                     W_lin1, b_lin1.reshape(1, HID),
                     W_lin2, b_lin2.reshape(1, OUT))


# speedup vs baseline: 93.4621x; 1.2374x over previous
import functools

import jax
import jax.numpy as jnp
from jax import lax
from jax.experimental import pallas as pl
from jax.experimental.pallas import tpu as pltpu
from jax.experimental.pallas import tpu_sc as plsc

N = 50000
E = 800000
IN_DIM = 10
HID = 64
OUT = 2
G = 128

F = 16
CH = 128
NCHE = E // CH
N_TILES = 32
CPT = NCHE // N_TILES
XTRA = NCHE - CPT * N_TILES
K_DEG = 13
G_DEG = CPT // K_DEG
K_AGG = 5
G_AGG = CPT // K_AGG

N2 = 51200
ROWS_T = N2 // 16
ZR = 800

R_TC = 2048
NT_TC = N2 // R_TC

_mesh = plsc.VectorSubcoreMesh(core_axis_name="c", subcore_axis_name="s")


def _stage_chunk(flat_v, stg_v, b, pos):
    for j in range(CH // 16):
        stg_v[b, pl.ds(j * 16, 16)] = flat_v[pl.ds(pos * CH + j * 16, 16)]


def _deg_body(ei_hbm, zeros_hbm, out_hbm, idxb_v, stg_v, ones_v, zbuf_v,
              deg_sh, sem):
    c = lax.axis_index("c")
    s = lax.axis_index("s")
    wid = c * 16 + s
    rbase = s * ROWS_T

    for t in range(ROWS_T // ZR):
        pltpu.sync_copy(zeros_hbm.at[pl.ds(rbase + t * ZR, ZR)], zbuf_v)
        pltpu.sync_copy(zbuf_v, deg_sh.at[pl.ds(rbase + t * ZR, ZR)])
    pltpu.sync_copy(ei_hbm.at[1, pl.ds(wid * CPT * CH, CPT * CH)], idxb_v)
    for j in range(CH // 16):
        ones_v[pl.ds(j * 16, 16)] = jnp.full((16,), 1.0, jnp.float32)
    plsc.subcore_barrier()

    def body(g, carry):
        for b in range(K_DEG):
            _stage_chunk(idxb_v, stg_v, b, g * K_DEG + b)
        hs = [pltpu.async_copy(ones_v, deg_sh.at[stg_v.at[b]], sem, add=True)
              for b in range(K_DEG)]
        for h in hs:
            h.wait()
        return carry

    lax.fori_loop(0, G_DEG, body, 0)

    @pl.when(wid < XTRA)
    def _():
        pltpu.sync_copy(ei_hbm.at[1, pl.ds((N_TILES * CPT + wid) * CH, CH)],
                        stg_v.at[0])
        pltpu.async_copy(ones_v, deg_sh.at[stg_v.at[0]], sem,
                         add=True).wait()

    plsc.subcore_barrier()
    for t in range(ROWS_T // ZR):
        pltpu.sync_copy(deg_sh.at[pl.ds(rbase + t * ZR, ZR)], zbuf_v)
        pltpu.sync_copy(zbuf_v, out_hbm.at[pl.ds(c * N2 + rbase + t * ZR, ZR)])


_deg_kernel = functools.partial(
    pl.kernel,
    out_type=jax.ShapeDtypeStruct((2 * N2,), jnp.float32),
    mesh=_mesh,
    scratch_types=[
        pltpu.VMEM((CPT * CH,), jnp.int32),
        pltpu.VMEM((K_DEG, CH), jnp.int32),
        pltpu.VMEM((CH,), jnp.float32),
        pltpu.VMEM((ZR,), jnp.float32),
        pltpu.VMEM_SHARED((N2,), jnp.float32),
        pltpu.SemaphoreType.DMA,
    ],
    compiler_params=pltpu.CompilerParams(use_tc_tiling_on_sc=False),
)(_deg_body)


def _agg_body(ei_hbm, xs_hbm, zeros_hbm, out_hbm,
              srcb_v, dstb_v, stg_v, rows_v, zbuf_v, acc_sh, gsem, ssem):
    c = lax.axis_index("c")
    s = lax.axis_index("s")
    wid = c * 16 + s
    rbase = s * ROWS_T

    for t in range(ROWS_T // ZR):
        pltpu.sync_copy(zeros_hbm.at[pl.ds(rbase + t * ZR, ZR)], zbuf_v)
        pltpu.sync_copy(zbuf_v, acc_sh.at[pl.ds(rbase + t * ZR, ZR)])
    pltpu.sync_copy(ei_hbm.at[0, pl.ds(wid * CPT * CH, CPT * CH)], srcb_v)
    pltpu.sync_copy(ei_hbm.at[1, pl.ds(wid * CPT * CH, CPT * CH)], dstb_v)
    plsc.subcore_barrier()

    def body(g, carry):
        gh = [pltpu.async_copy(
                  xs_hbm.at[srcb_v.at[pl.ds((g * K_AGG + b) * CH, CH)]],
                  rows_v.at[b], gsem)
              for b in range(K_AGG)]
        for b in range(K_AGG):
            _stage_chunk(dstb_v, stg_v, b, g * K_AGG + b)
        for h in gh:
            h.wait()
        sh = [pltpu.async_copy(rows_v.at[b], acc_sh.at[stg_v.at[b]],
                               ssem, add=True)
              for b in range(K_AGG)]
        for h in sh:
            h.wait()
        return carry

    lax.fori_loop(0, G_AGG, body, 0)

    @pl.when(wid < XTRA)
    def _():
        pltpu.sync_copy(ei_hbm.at[0, pl.ds((N_TILES * CPT + wid) * CH, CH)],
                        srcb_v.at[pl.ds(0, CH)])
        pltpu.sync_copy(ei_hbm.at[1, pl.ds((N_TILES * CPT + wid) * CH, CH)],
                        stg_v.at[0])
        pltpu.async_copy(xs_hbm.at[srcb_v.at[pl.ds(0, CH)]], rows_v.at[0],
                         gsem).wait()
        pltpu.async_copy(rows_v.at[0], acc_sh.at[stg_v.at[0]], ssem,
                         add=True).wait()

    plsc.subcore_barrier()
    for t in range(ROWS_T // ZR):
        pltpu.sync_copy(acc_sh.at[pl.ds(rbase + t * ZR, ZR)], zbuf_v)
        pltpu.sync_copy(zbuf_v, out_hbm.at[pl.ds(c * N2 + rbase + t * ZR, ZR)])


_agg_kernel = functools.partial(
    pl.kernel,
    out_type=jax.ShapeDtypeStruct((2 * N2, F), jnp.float32),
    mesh=_mesh,
    scratch_types=[
        pltpu.VMEM((CPT * CH,), jnp.int32),
        pltpu.VMEM((CPT * CH,), jnp.int32),
        pltpu.VMEM((K_AGG, CH), jnp.int32),
        pltpu.VMEM((K_AGG, CH, F), jnp.float32),
        pltpu.VMEM((ZR, F), jnp.float32),
        pltpu.VMEM_SHARED((N2, F), jnp.float32),
        pltpu.SemaphoreType.DMA,
        pltpu.SemaphoreType.DMA,
    ],
    compiler_params=pltpu.CompilerParams(use_tc_tiling_on_sc=False),
)(_agg_body)


RP = R_TC // 8
NP = N2 // 8


def _prep_body(d0_ref, d1_ref, x8_ref, xs_ref):
    dis8 = lax.rsqrt(d0_ref[...] + d1_ref[...] + 1.0)
    x8 = x8_ref[...]
    pieces = []
    for k in range(8):
        xk = dis8[:, k:k + 1] * x8[:, IN_DIM * k:IN_DIM * (k + 1)]
        pieces.append(xk)
        pieces.append(jnp.zeros((RP, F - IN_DIM), jnp.float32))
    xs_ref[...] = jnp.concatenate(pieces, axis=1)


def _prep_tc(deg8, x8):
    return pl.pallas_call(
        _prep_body,
        grid=(NT_TC,),
        in_specs=[
            pl.BlockSpec((RP, 8), lambda i: (i, 0)),
            pl.BlockSpec((RP, 8), lambda i: (i + NT_TC, 0)),
            pl.BlockSpec((RP, 8 * IN_DIM), lambda i: (i, 0)),
        ],
        out_specs=pl.BlockSpec((RP, 8 * F), lambda i: (i, 0)),
        out_shape=jax.ShapeDtypeStruct((NP, 8 * F), jnp.float32),
    )(deg8, deg8, x8)


def _dense_body(a0_ref, a1_ref, xs_ref, d0_ref, d1_ref, bat_ref, w1_ref,
                b1_ref, wl1_ref, bl1_ref, wl2_ref, bl2_ref, out_ref,
                acc, cnt):
    i = pl.program_id(0)

    @pl.when(i == 0)
    def _():
        acc[...] = jnp.zeros_like(acc)
        cnt[...] = jnp.zeros_like(cnt)

    dis8 = lax.rsqrt(d0_ref[...] + d1_ref[...] + 1.0)
    s8 = a0_ref[...] + a1_ref[...] + xs_ref[...]
    p8 = lax.dot_general(s8, w1_ref[...], (((1,), (0,)), ((), ())),
                         preferred_element_type=jnp.float32)
    gids = lax.broadcasted_iota(jnp.int32, (1, G), 1)
    bat8 = bat_ref[...]
    oh_sum = jnp.zeros((RP, G), jnp.float32)
    for k in range(8):
        hk = jnp.maximum(
            dis8[:, k:k + 1] * p8[:, HID * k:HID * (k + 1)] + b1_ref[...],
            0.0)
        ohk = (bat8[:, k:k + 1] == gids).astype(jnp.float32)
        oh_sum = oh_sum + ohk
        acc[...] += lax.dot_general(ohk, hk, (((0,), (0,)), ((), ())),
                                    preferred_element_type=jnp.float32)
    cnt[...] += lax.dot_general(oh_sum, jnp.ones((RP, 1), jnp.float32),
                                (((0,), (0,)), ((), ())),
                                preferred_element_type=jnp.float32)

    @pl.when(i == NT_TC - 1)
    def _():
        pooled = acc[...] / jnp.maximum(cnt[...], 1.0)
        z1 = jnp.maximum(
            lax.dot_general(pooled, wl1_ref[...], (((1,), (0,)), ((), ())),
                            preferred_element_type=jnp.float32)
            + bl1_ref[...], 0.0)
        z2 = lax.dot_general(z1, wl2_ref[...], (((1,), (0,)), ((), ())),
                             preferred_element_type=jnp.float32) + bl2_ref[...]
        m = jnp.max(z2, axis=1, keepdims=True)
        lse = m + jnp.log(jnp.sum(jnp.exp(z2 - m), axis=1, keepdims=True))
        out_ref[...] = z2 - lse


def _dense_tc(asum8, xs8, deg8, bat8, w1bd, b1, wl1, bl1, wl2, bl2):
    whole = lambda shape: pl.BlockSpec(shape, lambda i: tuple(0 for _ in shape))
    return pl.pallas_call(
        _dense_body,
        grid=(NT_TC,),
        in_specs=[
            pl.BlockSpec((RP, 8 * F), lambda i: (i, 0)),
            pl.BlockSpec((RP, 8 * F), lambda i: (i + NT_TC, 0)),
            pl.BlockSpec((RP, 8 * F), lambda i: (i, 0)),
            pl.BlockSpec((RP, 8), lambda i: (i, 0)),
            pl.BlockSpec((RP, 8), lambda i: (i + NT_TC, 0)),
            pl.BlockSpec((RP, 8), lambda i: (i, 0)),
            whole((8 * F, 8 * HID)),
            whole((1, HID)),
            whole((HID, HID)),
            whole((1, HID)),
            whole((HID, OUT)),
            whole((1, OUT)),
        ],
        out_specs=pl.BlockSpec((G, OUT), lambda i: (0, 0)),
        out_shape=jax.ShapeDtypeStruct((G, OUT), jnp.float32),
        scratch_shapes=[
            pltpu.VMEM((G, HID), jnp.float32),
            pltpu.VMEM((G, 1), jnp.float32),
        ],
    )(asum8, asum8, xs8, deg8, deg8, bat8, w1bd, b1, wl1, bl1, wl2, bl2)


def kernel(x, edge_index, batch, W1, b1, W_lin1, b_lin1, W_lin2, b_lin2):
    f32 = jnp.float32
    x8 = jnp.pad(x.reshape(N // 8, 8 * IN_DIM), ((0, NP - N // 8), (0, 0)))
    bat8 = jnp.pad(batch.reshape(N // 8, 8), ((0, NP - N // 8), (0, 0)),
                   constant_values=G)

    deg_p = _deg_kernel(edge_index, jnp.zeros((N2,), f32))
    deg8 = deg_p.reshape(2 * NP, 8)
    xs8 = _prep_tc(deg8, x8)
    asum_p = _agg_kernel(edge_index, xs8.reshape(N2, F),
                         jnp.zeros((N2, F), f32))

    w1bd = jnp.zeros((8 * F, 8 * HID), f32)
    for k in range(8):
        w1bd = w1bd.at[F * k:F * k + IN_DIM, HID * k:HID * (k + 1)].set(W1)
    return _dense_tc(asum_p.reshape(2 * NP, 8 * F), xs8, deg8, bat8,
                     w1bd, b1.reshape(1, HID),
                     W_lin1, b_lin1.reshape(1, HID),
                     W_lin2, b_lin2.reshape(1, OUT))


# speedup vs baseline: 96.6413x; 1.0340x over previous
import functools

import jax
import jax.numpy as jnp
from jax import lax
from jax.experimental import pallas as pl
from jax.experimental.pallas import tpu as pltpu
from jax.experimental.pallas import tpu_sc as plsc

N = 50000
E = 800000
IN_DIM = 10
HID = 64
OUT = 2
G = 128

F = 16
CH = 128
NCHE = E // CH
N_TILES = 32
CPT = NCHE // N_TILES
XTRA = NCHE - CPT * N_TILES
K_DEG = 13
G_DEG = CPT // K_DEG
K_AGG = 5
G_AGG = CPT // K_AGG

N2 = 51200
ROWS_T = N2 // 16
ZR = 400

R_TC = 2048
NT_TC = N2 // R_TC

_mesh = plsc.VectorSubcoreMesh(core_axis_name="c", subcore_axis_name="s")


def _stage_chunk(flat_v, stg_v, b, pos):
    for j in range(CH // 16):
        stg_v[b, pl.ds(j * 16, 16)] = flat_v[pl.ds(pos * CH + j * 16, 16)]


def _deg_body(ei_hbm, zeros_hbm, out_hbm, idxb_v, stg_v, ones_v, zbuf_v,
              deg_sh, sem):
    c = lax.axis_index("c")
    s = lax.axis_index("s")
    wid = c * 16 + s
    rbase = s * ROWS_T

    for t in range(ROWS_T // ZR):
        pltpu.sync_copy(zeros_hbm.at[pl.ds(rbase + t * ZR, ZR)], zbuf_v)
        pltpu.sync_copy(zbuf_v, deg_sh.at[pl.ds(rbase + t * ZR, ZR)])
    pltpu.sync_copy(ei_hbm.at[1, pl.ds(wid * CPT * CH, CPT * CH)], idxb_v)
    for j in range(CH // 16):
        ones_v[pl.ds(j * 16, 16)] = jnp.full((16,), 1.0, jnp.float32)
    plsc.subcore_barrier()

    def body(g, carry):
        for b in range(K_DEG):
            _stage_chunk(idxb_v, stg_v, b, g * K_DEG + b)
        hs = [pltpu.async_copy(ones_v, deg_sh.at[stg_v.at[b]], sem, add=True)
              for b in range(K_DEG)]
        for h in hs:
            h.wait()
        return carry

    lax.fori_loop(0, G_DEG, body, 0)

    @pl.when(wid < XTRA)
    def _():
        pltpu.sync_copy(ei_hbm.at[1, pl.ds((N_TILES * CPT + wid) * CH, CH)],
                        stg_v.at[0])
        pltpu.async_copy(ones_v, deg_sh.at[stg_v.at[0]], sem,
                         add=True).wait()

    plsc.subcore_barrier()
    for t in range(ROWS_T // ZR):
        pltpu.sync_copy(deg_sh.at[pl.ds(rbase + t * ZR, ZR)], zbuf_v)
        pltpu.sync_copy(zbuf_v, out_hbm.at[pl.ds(c * N2 + rbase + t * ZR, ZR)])


_deg_kernel = functools.partial(
    pl.kernel,
    out_type=jax.ShapeDtypeStruct((2 * N2,), jnp.float32),
    mesh=_mesh,
    scratch_types=[
        pltpu.VMEM((CPT * CH,), jnp.int32),
        pltpu.VMEM((K_DEG, CH), jnp.int32),
        pltpu.VMEM((CH,), jnp.float32),
        pltpu.VMEM((ZR,), jnp.float32),
        pltpu.VMEM_SHARED((N2,), jnp.float32),
        pltpu.SemaphoreType.DMA,
    ],
    compiler_params=pltpu.CompilerParams(use_tc_tiling_on_sc=False),
)(_deg_body)


def _agg_body(ei_hbm, xs_hbm, zeros_hbm, out_hbm,
              srcb_v, dstb_v, stg_v, rows_v, zbuf_v, acc_sh,
              gsem0, gsem1, ssem0, ssem1):
    c = lax.axis_index("c")
    s = lax.axis_index("s")
    wid = c * 16 + s
    rbase = s * ROWS_T
    gsem = [gsem0, gsem1]
    ssem = [ssem0, ssem1]

    for t in range(ROWS_T // ZR):
        pltpu.sync_copy(zeros_hbm.at[pl.ds(rbase + t * ZR, ZR)], zbuf_v)
        pltpu.sync_copy(zbuf_v, acc_sh.at[pl.ds(rbase + t * ZR, ZR)])
    pltpu.sync_copy(ei_hbm.at[0, pl.ds(wid * CPT * CH, CPT * CH)], srcb_v)
    pltpu.sync_copy(ei_hbm.at[1, pl.ds(wid * CPT * CH, CPT * CH)], dstb_v)
    plsc.subcore_barrier()

    def issue_gathers(g, h):
        return [pltpu.async_copy(
                    xs_hbm.at[srcb_v.at[pl.ds((g * K_AGG + b) * CH, CH)]],
                    rows_v.at[h, b], gsem[h])
                for b in range(K_AGG)]

    def wait_gathers(g, h):
        for b in range(K_AGG):
            pltpu.make_async_copy(
                xs_hbm.at[srcb_v.at[pl.ds((g * K_AGG + b) * CH, CH)]],
                rows_v.at[h, b], gsem[h]).wait()

    def issue_scatters(g, h):
        for b in range(K_AGG):
            for j in range(CH // 16):
                stg_v[h, b, pl.ds(j * 16, 16)] = \
                    dstb_v[pl.ds((g * K_AGG + b) * CH + j * 16, 16)]
        return [pltpu.async_copy(rows_v.at[h, b], acc_sh.at[stg_v.at[h, b]],
                                 ssem[h], add=True)
                for b in range(K_AGG)]

    def wait_scatters(g, h):
        for b in range(K_AGG):
            pltpu.make_async_copy(rows_v.at[h, b],
                                  acc_sh.at[stg_v.at[h, b]], ssem[h]).wait()

    issue_gathers(0, 0)

    def body(i, carry):
        wait_gathers(2 * i, 0)
        issue_scatters(2 * i, 0)

        @pl.when(i > 0)
        def _():
            wait_scatters(2 * i - 1, 1)

        issue_gathers(2 * i + 1, 1)
        wait_gathers(2 * i + 1, 1)
        issue_scatters(2 * i + 1, 1)
        wait_scatters(2 * i, 0)
        issue_gathers(2 * i + 2, 0)
        return carry

    lax.fori_loop(0, (G_AGG - 1) // 2, body, 0)
    wait_gathers(G_AGG - 1, 0)
    issue_scatters(G_AGG - 1, 0)
    wait_scatters(G_AGG - 2, 1)
    wait_scatters(G_AGG - 1, 0)

    @pl.when(wid < XTRA)
    def _():
        pltpu.sync_copy(ei_hbm.at[0, pl.ds((N_TILES * CPT + wid) * CH, CH)],
                        srcb_v.at[pl.ds(0, CH)])
        pltpu.sync_copy(ei_hbm.at[1, pl.ds((N_TILES * CPT + wid) * CH, CH)],
                        stg_v.at[0, 0])
        pltpu.async_copy(xs_hbm.at[srcb_v.at[pl.ds(0, CH)]],
                         rows_v.at[0, 0], gsem0).wait()
        pltpu.async_copy(rows_v.at[0, 0], acc_sh.at[stg_v.at[0, 0]], ssem0,
                         add=True).wait()

    plsc.subcore_barrier()
    for t in range(ROWS_T // ZR):
        pltpu.sync_copy(acc_sh.at[pl.ds(rbase + t * ZR, ZR)], zbuf_v)
        pltpu.sync_copy(zbuf_v, out_hbm.at[pl.ds(c * N2 + rbase + t * ZR, ZR)])


_agg_kernel = functools.partial(
    pl.kernel,
    out_type=jax.ShapeDtypeStruct((2 * N2, F), jnp.float32),
    mesh=_mesh,
    scratch_types=[
        pltpu.VMEM((CPT * CH,), jnp.int32),
        pltpu.VMEM((CPT * CH,), jnp.int32),
        pltpu.VMEM((2, K_AGG, CH), jnp.int32),
        pltpu.VMEM((2, K_AGG, CH, F), jnp.float32),
        pltpu.VMEM((ZR, F), jnp.float32),
        pltpu.VMEM_SHARED((N2, F), jnp.float32),
        pltpu.SemaphoreType.DMA,
        pltpu.SemaphoreType.DMA,
        pltpu.SemaphoreType.DMA,
        pltpu.SemaphoreType.DMA,
    ],
    compiler_params=pltpu.CompilerParams(use_tc_tiling_on_sc=False),
)(_agg_body)


RP = R_TC // 8
NP = N2 // 8


def _prep_body(d0_ref, d1_ref, x8_ref, xs_ref):
    dis8 = lax.rsqrt(d0_ref[...] + d1_ref[...] + 1.0)
    x8 = x8_ref[...]
    pieces = []
    for k in range(8):
        xk = dis8[:, k:k + 1] * x8[:, IN_DIM * k:IN_DIM * (k + 1)]
        pieces.append(xk)
        pieces.append(jnp.zeros((RP, F - IN_DIM), jnp.float32))
    xs_ref[...] = jnp.concatenate(pieces, axis=1)


def _prep_tc(deg8, x8):
    return pl.pallas_call(
        _prep_body,
        grid=(NT_TC,),
        in_specs=[
            pl.BlockSpec((RP, 8), lambda i: (i, 0)),
            pl.BlockSpec((RP, 8), lambda i: (i + NT_TC, 0)),
            pl.BlockSpec((RP, 8 * IN_DIM), lambda i: (i, 0)),
        ],
        out_specs=pl.BlockSpec((RP, 8 * F), lambda i: (i, 0)),
        out_shape=jax.ShapeDtypeStruct((NP, 8 * F), jnp.float32),
    )(deg8, deg8, x8)


def _dense_body(a0_ref, a1_ref, xs_ref, d0_ref, d1_ref, bat_ref, w1_ref,
                b1_ref, wl1_ref, bl1_ref, wl2_ref, bl2_ref, out_ref,
                acc, cnt):
    i = pl.program_id(0)

    @pl.when(i == 0)
    def _():
        acc[...] = jnp.zeros_like(acc)
        cnt[...] = jnp.zeros_like(cnt)

    dis8 = lax.rsqrt(d0_ref[...] + d1_ref[...] + 1.0)
    s8 = a0_ref[...] + a1_ref[...] + xs_ref[...]
    p8 = lax.dot_general(s8, w1_ref[...], (((1,), (0,)), ((), ())),
                         preferred_element_type=jnp.float32)
    gids = lax.broadcasted_iota(jnp.int32, (1, G), 1)
    bat8 = bat_ref[...]
    oh_sum = jnp.zeros((RP, G), jnp.float32)
    for k in range(8):
        hk = jnp.maximum(
            dis8[:, k:k + 1] * p8[:, HID * k:HID * (k + 1)] + b1_ref[...],
            0.0)
        ohk = (bat8[:, k:k + 1] == gids).astype(jnp.float32)
        oh_sum = oh_sum + ohk
        acc[...] += lax.dot_general(ohk, hk, (((0,), (0,)), ((), ())),
                                    preferred_element_type=jnp.float32)
    cnt[...] += lax.dot_general(oh_sum, jnp.ones((RP, 1), jnp.float32),
                                (((0,), (0,)), ((), ())),
                                preferred_element_type=jnp.float32)

    @pl.when(i == NT_TC - 1)
    def _():
        pooled = acc[...] / jnp.maximum(cnt[...], 1.0)
        z1 = jnp.maximum(
            lax.dot_general(pooled, wl1_ref[...], (((1,), (0,)), ((), ())),
                            preferred_element_type=jnp.float32)
            + bl1_ref[...], 0.0)
        z2 = lax.dot_general(z1, wl2_ref[...], (((1,), (0,)), ((), ())),
                             preferred_element_type=jnp.float32) + bl2_ref[...]
        m = jnp.max(z2, axis=1, keepdims=True)
        lse = m + jnp.log(jnp.sum(jnp.exp(z2 - m), axis=1, keepdims=True))
        out_ref[...] = z2 - lse


def _dense_tc(asum8, xs8, deg8, bat8, w1bd, b1, wl1, bl1, wl2, bl2):
    whole = lambda shape: pl.BlockSpec(shape, lambda i: tuple(0 for _ in shape))
    return pl.pallas_call(
        _dense_body,
        grid=(NT_TC,),
        in_specs=[
            pl.BlockSpec((RP, 8 * F), lambda i: (i, 0)),
            pl.BlockSpec((RP, 8 * F), lambda i: (i + NT_TC, 0)),
            pl.BlockSpec((RP, 8 * F), lambda i: (i, 0)),
            pl.BlockSpec((RP, 8), lambda i: (i, 0)),
            pl.BlockSpec((RP, 8), lambda i: (i + NT_TC, 0)),
            pl.BlockSpec((RP, 8), lambda i: (i, 0)),
            whole((8 * F, 8 * HID)),
            whole((1, HID)),
            whole((HID, HID)),
            whole((1, HID)),
            whole((HID, OUT)),
            whole((1, OUT)),
        ],
        out_specs=pl.BlockSpec((G, OUT), lambda i: (0, 0)),
        out_shape=jax.ShapeDtypeStruct((G, OUT), jnp.float32),
        scratch_shapes=[
            pltpu.VMEM((G, HID), jnp.float32),
            pltpu.VMEM((G, 1), jnp.float32),
        ],
    )(asum8, asum8, xs8, deg8, deg8, bat8, w1bd, b1, wl1, bl1, wl2, bl2)


def kernel(x, edge_index, batch, W1, b1, W_lin1, b_lin1, W_lin2, b_lin2):
    f32 = jnp.float32
    x8 = jnp.pad(x.reshape(N // 8, 8 * IN_DIM), ((0, NP - N // 8), (0, 0)))
    bat8 = jnp.pad(batch.reshape(N // 8, 8), ((0, NP - N // 8), (0, 0)),
                   constant_values=G)

    deg_p = _deg_kernel(edge_index, jnp.zeros((N2,), f32))
    deg8 = deg_p.reshape(2 * NP, 8)
    xs8 = _prep_tc(deg8, x8)
    asum_p = _agg_kernel(edge_index, xs8.reshape(N2, F),
                         jnp.zeros((N2, F), f32))

    w1bd = jnp.zeros((8 * F, 8 * HID), f32)
    for k in range(8):
        w1bd = w1bd.at[F * k:F * k + IN_DIM, HID * k:HID * (k + 1)].set(W1)
    return _dense_tc(asum_p.reshape(2 * NP, 8 * F), xs8, deg8, bat8,
                     w1bd, b1.reshape(1, HID),
                     W_lin1, b_lin1.reshape(1, HID),
                     W_lin2, b_lin2.reshape(1, OUT))


# speedup vs baseline: 106.3168x; 1.1001x over previous
import functools

import jax
import jax.numpy as jnp
from jax import lax
from jax.experimental import pallas as pl
from jax.experimental.pallas import tpu as pltpu
from jax.experimental.pallas import tpu_sc as plsc

N = 50000
E = 800000
IN_DIM = 10
HID = 64
OUT = 2
G = 128

F = 16
CH = 128
NCHE = E // CH
N_TILES = 32
CPT = NCHE // N_TILES
XTRA = NCHE - CPT * N_TILES
K_DEG = 13
G_DEG = CPT // K_DEG
K_AGG = 5
G_AGG = CPT // K_AGG

N2 = 51200
ROWS_T = N2 // 16
ZR = 400

R_TC = 5120
NT_TC = N2 // R_TC

_mesh = plsc.VectorSubcoreMesh(core_axis_name="c", subcore_axis_name="s")


def _stage_chunk(flat_v, stg_v, b, pos):
    for j in range(CH // 16):
        stg_v[b, pl.ds(j * 16, 16)] = flat_v[pl.ds(pos * CH + j * 16, 16)]


def _deg_body(ei_hbm, zeros_hbm, out_hbm, idxb_v, stg_v, ones_v, zbuf_v,
              deg_sh, sem):
    c = lax.axis_index("c")
    s = lax.axis_index("s")
    wid = c * 16 + s
    rbase = s * ROWS_T

    for t in range(ROWS_T // ZR):
        pltpu.sync_copy(zeros_hbm.at[pl.ds(rbase + t * ZR, ZR)], zbuf_v)
        pltpu.sync_copy(zbuf_v, deg_sh.at[pl.ds(rbase + t * ZR, ZR)])
    pltpu.sync_copy(ei_hbm.at[1, pl.ds(wid * CPT * CH, CPT * CH)], idxb_v)
    for j in range(CH // 16):
        ones_v[pl.ds(j * 16, 16)] = jnp.full((16,), 1.0, jnp.float32)
    plsc.subcore_barrier()

    def body(g, carry):
        for b in range(K_DEG):
            _stage_chunk(idxb_v, stg_v, b, g * K_DEG + b)
        hs = [pltpu.async_copy(ones_v, deg_sh.at[stg_v.at[b]], sem, add=True)
              for b in range(K_DEG)]
        for h in hs:
            h.wait()
        return carry

    lax.fori_loop(0, G_DEG, body, 0)

    @pl.when(wid < XTRA)
    def _():
        pltpu.sync_copy(ei_hbm.at[1, pl.ds((N_TILES * CPT + wid) * CH, CH)],
                        stg_v.at[0])
        pltpu.async_copy(ones_v, deg_sh.at[stg_v.at[0]], sem,
                         add=True).wait()

    plsc.subcore_barrier()
    for t in range(ROWS_T // ZR):
        pltpu.sync_copy(deg_sh.at[pl.ds(rbase + t * ZR, ZR)], zbuf_v)
        pltpu.sync_copy(zbuf_v, out_hbm.at[pl.ds(c * N2 + rbase + t * ZR, ZR)])


_deg_kernel = functools.partial(
    pl.kernel,
    out_type=jax.ShapeDtypeStruct((2 * N2,), jnp.float32),
    mesh=_mesh,
    scratch_types=[
        pltpu.VMEM((CPT * CH,), jnp.int32),
        pltpu.VMEM((K_DEG, CH), jnp.int32),
        pltpu.VMEM((CH,), jnp.float32),
        pltpu.VMEM((ZR,), jnp.float32),
        pltpu.VMEM_SHARED((N2,), jnp.float32),
        pltpu.SemaphoreType.DMA,
    ],
    compiler_params=pltpu.CompilerParams(use_tc_tiling_on_sc=False),
)(_deg_body)


def _agg_body(ei_hbm, xs_hbm, zeros_hbm, out_hbm,
              srcb_v, dstb_v, stg_v, rows_v, zbuf_v, acc_sh,
              gsem0, gsem1, ssem0, ssem1):
    c = lax.axis_index("c")
    s = lax.axis_index("s")
    wid = c * 16 + s
    rbase = s * ROWS_T
    gsem = [gsem0, gsem1]
    ssem = [ssem0, ssem1]

    for t in range(ROWS_T // ZR):
        pltpu.sync_copy(zeros_hbm.at[pl.ds(rbase + t * ZR, ZR)], zbuf_v)
        pltpu.sync_copy(zbuf_v, acc_sh.at[pl.ds(rbase + t * ZR, ZR)])
    pltpu.sync_copy(ei_hbm.at[0, pl.ds(wid * CPT * CH, CPT * CH)], srcb_v)
    pltpu.sync_copy(ei_hbm.at[1, pl.ds(wid * CPT * CH, CPT * CH)], dstb_v)
    plsc.subcore_barrier()

    def issue_gathers(g, h):
        return [pltpu.async_copy(
                    xs_hbm.at[srcb_v.at[pl.ds((g * K_AGG + b) * CH, CH)]],
                    rows_v.at[h, b], gsem[h])
                for b in range(K_AGG)]

    def wait_gathers(g, h):
        for b in range(K_AGG):
            pltpu.make_async_copy(
                xs_hbm.at[srcb_v.at[pl.ds((g * K_AGG + b) * CH, CH)]],
                rows_v.at[h, b], gsem[h]).wait()

    def issue_scatters(g, h):
        for b in range(K_AGG):
            for j in range(CH // 16):
                stg_v[h, b, pl.ds(j * 16, 16)] = \
                    dstb_v[pl.ds((g * K_AGG + b) * CH + j * 16, 16)]
        return [pltpu.async_copy(rows_v.at[h, b], acc_sh.at[stg_v.at[h, b]],
                                 ssem[h], add=True)
                for b in range(K_AGG)]

    def wait_scatters(g, h):
        for b in range(K_AGG):
            pltpu.make_async_copy(rows_v.at[h, b],
                                  acc_sh.at[stg_v.at[h, b]], ssem[h]).wait()

    issue_gathers(0, 0)

    def body(i, carry):
        wait_gathers(2 * i, 0)
        issue_scatters(2 * i, 0)

        @pl.when(i > 0)
        def _():
            wait_scatters(2 * i - 1, 1)

        issue_gathers(2 * i + 1, 1)
        wait_gathers(2 * i + 1, 1)
        issue_scatters(2 * i + 1, 1)
        wait_scatters(2 * i, 0)
        issue_gathers(2 * i + 2, 0)
        return carry

    lax.fori_loop(0, (G_AGG - 1) // 2, body, 0)
    wait_gathers(G_AGG - 1, 0)
    issue_scatters(G_AGG - 1, 0)
    wait_scatters(G_AGG - 2, 1)
    wait_scatters(G_AGG - 1, 0)

    @pl.when(wid < XTRA)
    def _():
        pltpu.sync_copy(ei_hbm.at[0, pl.ds((N_TILES * CPT + wid) * CH, CH)],
                        srcb_v.at[pl.ds(0, CH)])
        pltpu.sync_copy(ei_hbm.at[1, pl.ds((N_TILES * CPT + wid) * CH, CH)],
                        stg_v.at[0, 0])
        pltpu.async_copy(xs_hbm.at[srcb_v.at[pl.ds(0, CH)]],
                         rows_v.at[0, 0], gsem0).wait()
        pltpu.async_copy(rows_v.at[0, 0], acc_sh.at[stg_v.at[0, 0]], ssem0,
                         add=True).wait()

    plsc.subcore_barrier()
    for t in range(ROWS_T // ZR):
        pltpu.sync_copy(acc_sh.at[pl.ds(rbase + t * ZR, ZR)], zbuf_v)
        pltpu.sync_copy(zbuf_v, out_hbm.at[pl.ds(c * N2 + rbase + t * ZR, ZR)])


_agg_kernel = functools.partial(
    pl.kernel,
    out_type=jax.ShapeDtypeStruct((2 * N2, F), jnp.float32),
    mesh=_mesh,
    scratch_types=[
        pltpu.VMEM((CPT * CH,), jnp.int32),
        pltpu.VMEM((CPT * CH,), jnp.int32),
        pltpu.VMEM((2, K_AGG, CH), jnp.int32),
        pltpu.VMEM((2, K_AGG, CH, F), jnp.float32),
        pltpu.VMEM((ZR, F), jnp.float32),
        pltpu.VMEM_SHARED((N2, F), jnp.float32),
        pltpu.SemaphoreType.DMA,
        pltpu.SemaphoreType.DMA,
        pltpu.SemaphoreType.DMA,
        pltpu.SemaphoreType.DMA,
    ],
    compiler_params=pltpu.CompilerParams(use_tc_tiling_on_sc=False),
)(_agg_body)


RP = R_TC // 8
NP = N2 // 8


def _prep_body(d0_ref, d1_ref, x8_ref, rep16_ref, perm_ref, xs_ref):
    dis8 = lax.rsqrt(d0_ref[...] + d1_ref[...] + 1.0)
    dis16 = lax.dot_general(dis8, rep16_ref[...], (((1,), (0,)), ((), ())),
                            preferred_element_type=jnp.float32)
    x16 = lax.dot_general(x8_ref[...], perm_ref[...], (((1,), (0,)), ((), ())),
                          preferred_element_type=jnp.float32)
    xs_ref[...] = dis16 * x16


def _prep_tc(deg8, x8, rep16, perm):
    whole = lambda shape: pl.BlockSpec(shape, lambda i: tuple(0 for _ in shape))
    return pl.pallas_call(
        _prep_body,
        grid=(NT_TC,),
        in_specs=[
            pl.BlockSpec((RP, 8), lambda i: (i, 0)),
            pl.BlockSpec((RP, 8), lambda i: (i + NT_TC, 0)),
            pl.BlockSpec((RP, 8 * IN_DIM), lambda i: (i, 0)),
            whole((8, 8 * F)),
            whole((8 * IN_DIM, 8 * F)),
        ],
        out_specs=pl.BlockSpec((RP, 8 * F), lambda i: (i, 0)),
        out_shape=jax.ShapeDtypeStruct((NP, 8 * F), jnp.float32),
    )(deg8, deg8, x8, rep16, perm)


def _dense_body(a0_ref, a1_ref, xs_ref, d0_ref, d1_ref, bat_ref, w1_ref,
                rep64_ref, b1r_ref, b1_ref, wl1_ref, bl1_ref, wl2_ref,
                bl2_ref, out_ref, acc, cnt):
    i = pl.program_id(0)

    @pl.when(i == 0)
    def _():
        acc[...] = jnp.zeros_like(acc)
        cnt[...] = jnp.zeros_like(cnt)

    dis8 = lax.rsqrt(d0_ref[...] + d1_ref[...] + 1.0)
    dis64 = lax.dot_general(dis8, rep64_ref[...], (((1,), (0,)), ((), ())),
                            preferred_element_type=jnp.float32)
    s8 = a0_ref[...] + a1_ref[...] + xs_ref[...]
    p8 = lax.dot_general(s8, w1_ref[...], (((1,), (0,)), ((), ())),
                         preferred_element_type=jnp.float32)
    h8 = jnp.maximum(dis64 * p8 + b1r_ref[...], 0.0)
    gids = lax.broadcasted_iota(jnp.int32, (1, G), 1)
    bat8 = bat_ref[...]
    oh_sum = jnp.zeros((RP, G), jnp.float32)
    for k in range(8):
        ohk = (bat8[:, k:k + 1] == gids).astype(jnp.float32)
        oh_sum = oh_sum + ohk
        acc[...] += lax.dot_general(ohk, h8[:, HID * k:HID * (k + 1)],
                                    (((0,), (0,)), ((), ())),
                                    preferred_element_type=jnp.float32)
    cnt[...] += lax.dot_general(oh_sum, jnp.ones((RP, 1), jnp.float32),
                                (((0,), (0,)), ((), ())),
                                preferred_element_type=jnp.float32)

    @pl.when(i == NT_TC - 1)
    def _():
        pooled = acc[...] / jnp.maximum(cnt[...], 1.0)
        z1 = jnp.maximum(
            lax.dot_general(pooled, wl1_ref[...], (((1,), (0,)), ((), ())),
                            preferred_element_type=jnp.float32)
            + bl1_ref[...], 0.0)
        z2 = lax.dot_general(z1, wl2_ref[...], (((1,), (0,)), ((), ())),
                             preferred_element_type=jnp.float32) + bl2_ref[...]
        m = jnp.max(z2, axis=1, keepdims=True)
        lse = m + jnp.log(jnp.sum(jnp.exp(z2 - m), axis=1, keepdims=True))
        out_ref[...] = z2 - lse


def _dense_tc(asum8, xs8, deg8, bat8, w1bd, rep64, b1rep, b1, wl1, bl1,
              wl2, bl2):
    whole = lambda shape: pl.BlockSpec(shape, lambda i: tuple(0 for _ in shape))
    return pl.pallas_call(
        _dense_body,
        grid=(NT_TC,),
        in_specs=[
            pl.BlockSpec((RP, 8 * F), lambda i: (i, 0)),
            pl.BlockSpec((RP, 8 * F), lambda i: (i + NT_TC, 0)),
            pl.BlockSpec((RP, 8 * F), lambda i: (i, 0)),
            pl.BlockSpec((RP, 8), lambda i: (i, 0)),
            pl.BlockSpec((RP, 8), lambda i: (i + NT_TC, 0)),
            pl.BlockSpec((RP, 8), lambda i: (i, 0)),
            whole((8 * F, 8 * HID)),
            whole((8, 8 * HID)),
            whole((1, 8 * HID)),
            whole((1, HID)),
            whole((HID, HID)),
            whole((1, HID)),
            whole((HID, OUT)),
            whole((1, OUT)),
        ],
        out_specs=pl.BlockSpec((G, OUT), lambda i: (0, 0)),
        out_shape=jax.ShapeDtypeStruct((G, OUT), jnp.float32),
        scratch_shapes=[
            pltpu.VMEM((G, HID), jnp.float32),
            pltpu.VMEM((G, 1), jnp.float32),
        ],
    )(asum8, asum8, xs8, deg8, deg8, bat8, w1bd, rep64, b1rep, b1,
      wl1, bl1, wl2, bl2)


def kernel(x, edge_index, batch, W1, b1, W_lin1, b_lin1, W_lin2, b_lin2):
    f32 = jnp.float32
    x8 = jnp.pad(x.reshape(N // 8, 8 * IN_DIM), ((0, NP - N // 8), (0, 0)))
    bat8 = jnp.pad(batch.reshape(N // 8, 8), ((0, NP - N // 8), (0, 0)),
                   constant_values=G)

    kk = jnp.arange(8)
    rep16 = (jnp.arange(8 * F)[None, :] // F == kk[:, None]).astype(f32)
    rep64 = (jnp.arange(8 * HID)[None, :] // HID == kk[:, None]).astype(f32)
    jj = jnp.arange(8 * IN_DIM)
    perm = jnp.zeros((8 * IN_DIM, 8 * F), f32).at[
        jj, (jj // IN_DIM) * F + jj % IN_DIM].set(1.0)
    b1rep = jnp.tile(b1, 8).reshape(1, 8 * HID)

    deg_p = _deg_kernel(edge_index, jnp.zeros((N2,), f32))
    deg8 = deg_p.reshape(2 * NP, 8)
    xs8 = _prep_tc(deg8, x8, rep16, perm)
    asum_p = _agg_kernel(edge_index, xs8.reshape(N2, F),
                         jnp.zeros((N2, F), f32))

    w1bd = jnp.zeros((8 * F, 8 * HID), f32)
    for k in range(8):
        w1bd = w1bd.at[F * k:F * k + IN_DIM, HID * k:HID * (k + 1)].set(W1)
    return _dense_tc(asum_p.reshape(2 * NP, 8 * F), xs8, deg8, bat8,
                     w1bd, rep64, b1rep, b1.reshape(1, HID),
                     W_lin1, b_lin1.reshape(1, HID),
                     W_lin2, b_lin2.reshape(1, OUT))


# speedup vs baseline: 108.9023x; 1.0243x over previous
import functools

import jax
import jax.numpy as jnp
from jax import lax
from jax.experimental import pallas as pl
from jax.experimental.pallas import tpu as pltpu
from jax.experimental.pallas import tpu_sc as plsc

N = 50000
E = 800000
IN_DIM = 10
HID = 64
OUT = 2
G = 128

F = 16
CH = 128
NCHE = E // CH
N_TILES = 32
CPT = NCHE // N_TILES
XTRA = NCHE - CPT * N_TILES
K_DEG = 13
G_DEG = CPT // K_DEG
K_AGG = 5
G_AGG = CPT // K_AGG

N2 = 51200
ROWS_T = N2 // 16
ZR = 400

R_TC = 5120
NT_TC = N2 // R_TC

_mesh = plsc.VectorSubcoreMesh(core_axis_name="c", subcore_axis_name="s")


DZR = 6400


def _deg_body(ei_hbm, out_hbm, idxb_v, stg_v, ones_v, zbuf_v, deg_sh, sem):
    c = lax.axis_index("c")
    s = lax.axis_index("s")
    wid = c * 16 + s
    wbase = s * (16 * ROWS_T)

    for j in range(DZR // 16):
        zbuf_v[pl.ds(j * 16, 16)] = jnp.zeros((16,), jnp.float32)
    for t in range(16 * ROWS_T // DZR):
        pltpu.sync_copy(zbuf_v, deg_sh.at[pl.ds(wbase + t * DZR, DZR)])
    pltpu.sync_copy(ei_hbm.at[1, pl.ds(wid * CPT * CH, CPT * CH)], idxb_v)
    for j in range(CH // 16):
        ones_v[pl.ds(j * 16, 16)] = jnp.full((16,), 1.0, jnp.float32)
    plsc.subcore_barrier()

    def stage16(b, pos):
        for j in range(CH // 16):
            stg_v[b, pl.ds(j * 16, 16)] = lax.shift_left(
                idxb_v[pl.ds(pos * CH + j * 16, 16)], 4)

    def body(g, carry):
        for b in range(K_DEG):
            stage16(b, g * K_DEG + b)
        hs = [pltpu.async_copy(ones_v, deg_sh.at[stg_v.at[b]], sem, add=True)
              for b in range(K_DEG)]
        for h in hs:
            h.wait()
        return carry

    lax.fori_loop(0, G_DEG, body, 0)

    @pl.when(wid < XTRA)
    def _():
        pltpu.sync_copy(ei_hbm.at[1, pl.ds((N_TILES * CPT + wid) * CH, CH)],
                        idxb_v.at[pl.ds(0, CH)])
        stage16(0, 0)
        pltpu.async_copy(ones_v, deg_sh.at[stg_v.at[0]], sem,
                         add=True).wait()

    plsc.subcore_barrier()
    for t in range(16 * ROWS_T // DZR):
        pltpu.sync_copy(deg_sh.at[pl.ds(wbase + t * DZR, DZR)], zbuf_v)
        pltpu.sync_copy(zbuf_v, out_hbm.at[pl.ds(c * 16 * N2 + wbase
                                                 + t * DZR, DZR)])


_deg_kernel = functools.partial(
    pl.kernel,
    out_type=jax.ShapeDtypeStruct((2 * 16 * N2,), jnp.float32),
    mesh=_mesh,
    scratch_types=[
        pltpu.VMEM((CPT * CH,), jnp.int32),
        pltpu.VMEM((K_DEG, CH), jnp.int32),
        pltpu.VMEM((CH,), jnp.float32),
        pltpu.VMEM((DZR,), jnp.float32),
        pltpu.VMEM_SHARED((16 * N2,), jnp.float32),
        pltpu.SemaphoreType.DMA,
    ],
    compiler_params=pltpu.CompilerParams(use_tc_tiling_on_sc=False),
)(_deg_body)


def _agg_body(ei_hbm, xs_hbm, zeros_hbm, out_hbm,
              srcb_v, dstb_v, stg_v, rows_v, zbuf_v, acc_sh,
              gsem0, gsem1, ssem0, ssem1):
    c = lax.axis_index("c")
    s = lax.axis_index("s")
    wid = c * 16 + s
    rbase = s * ROWS_T
    gsem = [gsem0, gsem1]
    ssem = [ssem0, ssem1]

    for t in range(ROWS_T // ZR):
        pltpu.sync_copy(zeros_hbm.at[pl.ds(rbase + t * ZR, ZR)], zbuf_v)
        pltpu.sync_copy(zbuf_v, acc_sh.at[pl.ds(rbase + t * ZR, ZR)])
    pltpu.sync_copy(ei_hbm.at[0, pl.ds(wid * CPT * CH, CPT * CH)], srcb_v)
    pltpu.sync_copy(ei_hbm.at[1, pl.ds(wid * CPT * CH, CPT * CH)], dstb_v)
    plsc.subcore_barrier()

    def issue_gathers(g, h):
        return [pltpu.async_copy(
                    xs_hbm.at[srcb_v.at[pl.ds((g * K_AGG + b) * CH, CH)]],
                    rows_v.at[h, b], gsem[h])
                for b in range(K_AGG)]

    def wait_gathers(g, h):
        for b in range(K_AGG):
            pltpu.make_async_copy(
                xs_hbm.at[srcb_v.at[pl.ds((g * K_AGG + b) * CH, CH)]],
                rows_v.at[h, b], gsem[h]).wait()

    def issue_scatters(g, h):
        for b in range(K_AGG):
            for j in range(CH // 16):
                stg_v[h, b, pl.ds(j * 16, 16)] = \
                    dstb_v[pl.ds((g * K_AGG + b) * CH + j * 16, 16)]
        return [pltpu.async_copy(rows_v.at[h, b], acc_sh.at[stg_v.at[h, b]],
                                 ssem[h], add=True)
                for b in range(K_AGG)]

    def wait_scatters(g, h):
        for b in range(K_AGG):
            pltpu.make_async_copy(rows_v.at[h, b],
                                  acc_sh.at[stg_v.at[h, b]], ssem[h]).wait()

    issue_gathers(0, 0)

    def body(i, carry):
        wait_gathers(2 * i, 0)
        issue_scatters(2 * i, 0)

        @pl.when(i > 0)
        def _():
            wait_scatters(2 * i - 1, 1)

        issue_gathers(2 * i + 1, 1)
        wait_gathers(2 * i + 1, 1)
        issue_scatters(2 * i + 1, 1)
        wait_scatters(2 * i, 0)
        issue_gathers(2 * i + 2, 0)
        return carry

    lax.fori_loop(0, (G_AGG - 1) // 2, body, 0)
    wait_gathers(G_AGG - 1, 0)
    issue_scatters(G_AGG - 1, 0)
    wait_scatters(G_AGG - 2, 1)
    wait_scatters(G_AGG - 1, 0)

    @pl.when(wid < XTRA)
    def _():
        pltpu.sync_copy(ei_hbm.at[0, pl.ds((N_TILES * CPT + wid) * CH, CH)],
                        srcb_v.at[pl.ds(0, CH)])
        pltpu.sync_copy(ei_hbm.at[1, pl.ds((N_TILES * CPT + wid) * CH, CH)],
                        stg_v.at[0, 0])
        pltpu.async_copy(xs_hbm.at[srcb_v.at[pl.ds(0, CH)]],
                         rows_v.at[0, 0], gsem0).wait()
        pltpu.async_copy(rows_v.at[0, 0], acc_sh.at[stg_v.at[0, 0]], ssem0,
                         add=True).wait()

    plsc.subcore_barrier()
    for t in range(ROWS_T // ZR):
        pltpu.sync_copy(acc_sh.at[pl.ds(rbase + t * ZR, ZR)], zbuf_v)
        pltpu.sync_copy(zbuf_v, out_hbm.at[pl.ds(c * N2 + rbase + t * ZR, ZR)])


_agg_kernel = functools.partial(
    pl.kernel,
    out_type=jax.ShapeDtypeStruct((2 * N2, F), jnp.float32),
    mesh=_mesh,
    scratch_types=[
        pltpu.VMEM((CPT * CH,), jnp.int32),
        pltpu.VMEM((CPT * CH,), jnp.int32),
        pltpu.VMEM((2, K_AGG, CH), jnp.int32),
        pltpu.VMEM((2, K_AGG, CH, F), jnp.float32),
        pltpu.VMEM((ZR, F), jnp.float32),
        pltpu.VMEM_SHARED((N2, F), jnp.float32),
        pltpu.SemaphoreType.DMA,
        pltpu.SemaphoreType.DMA,
        pltpu.SemaphoreType.DMA,
        pltpu.SemaphoreType.DMA,
    ],
    compiler_params=pltpu.CompilerParams(use_tc_tiling_on_sc=False),
)(_agg_body)


RP = R_TC // 8
NP = N2 // 8


def _prep_body(d0_ref, d1_ref, x8_ref, spread_ref, perm_ref, xs_ref):
    dsum = d0_ref[...] + d1_ref[...]
    dis16 = lax.rsqrt(
        lax.dot_general(dsum, spread_ref[...], (((1,), (0,)), ((), ())),
                        preferred_element_type=jnp.float32) + 1.0)
    x16 = lax.dot_general(x8_ref[...], perm_ref[...], (((1,), (0,)), ((), ())),
                          preferred_element_type=jnp.float32)
    xs_ref[...] = dis16 * x16


def _prep_tc(deg16, x8, spread16, perm):
    whole = lambda shape: pl.BlockSpec(shape, lambda i: tuple(0 for _ in shape))
    return pl.pallas_call(
        _prep_body,
        grid=(NT_TC,),
        in_specs=[
            pl.BlockSpec((RP, 8 * F), lambda i: (i, 0)),
            pl.BlockSpec((RP, 8 * F), lambda i: (i + NT_TC, 0)),
            pl.BlockSpec((RP, 8 * IN_DIM), lambda i: (i, 0)),
            whole((8 * F, 8 * F)),
            whole((8 * IN_DIM, 8 * F)),
        ],
        out_specs=pl.BlockSpec((RP, 8 * F), lambda i: (i, 0)),
        out_shape=jax.ShapeDtypeStruct((NP, 8 * F), jnp.float32),
    )(deg16, deg16, x8, spread16, perm)


def _dense_body(a0_ref, a1_ref, xs_ref, d0_ref, d1_ref, bat_ref, w1_ref,
                rep64_ref, b1r_ref, b1_ref, wl1_ref, bl1_ref, wl2_ref,
                bl2_ref, out_ref, acc, cnt):
    i = pl.program_id(0)

    @pl.when(i == 0)
    def _():
        acc[...] = jnp.zeros_like(acc)
        cnt[...] = jnp.zeros_like(cnt)

    dsum = d0_ref[...] + d1_ref[...]
    dis64 = lax.rsqrt(
        lax.dot_general(dsum, rep64_ref[...], (((1,), (0,)), ((), ())),
                        preferred_element_type=jnp.float32) + 1.0)
    s8 = a0_ref[...] + a1_ref[...] + xs_ref[...]
    p8 = lax.dot_general(s8, w1_ref[...], (((1,), (0,)), ((), ())),
                         preferred_element_type=jnp.float32)
    h8 = jnp.maximum(dis64 * p8 + b1r_ref[...], 0.0)
    gids = lax.broadcasted_iota(jnp.int32, (1, G), 1)
    bat8 = bat_ref[...]
    oh_sum = jnp.zeros((RP, G), jnp.float32)
    for k in range(8):
        ohk = (bat8[:, k:k + 1] == gids).astype(jnp.float32)
        oh_sum = oh_sum + ohk
        acc[...] += lax.dot_general(ohk, h8[:, HID * k:HID * (k + 1)],
                                    (((0,), (0,)), ((), ())),
                                    preferred_element_type=jnp.float32)
    cnt[...] += lax.dot_general(oh_sum, jnp.ones((RP, 1), jnp.float32),
                                (((0,), (0,)), ((), ())),
                                preferred_element_type=jnp.float32)

    @pl.when(i == NT_TC - 1)
    def _():
        pooled = acc[...] / jnp.maximum(cnt[...], 1.0)
        z1 = jnp.maximum(
            lax.dot_general(pooled, wl1_ref[...], (((1,), (0,)), ((), ())),
                            preferred_element_type=jnp.float32)
            + bl1_ref[...], 0.0)
        z2 = lax.dot_general(z1, wl2_ref[...], (((1,), (0,)), ((), ())),
                             preferred_element_type=jnp.float32) + bl2_ref[...]
        m = jnp.max(z2, axis=1, keepdims=True)
        lse = m + jnp.log(jnp.sum(jnp.exp(z2 - m), axis=1, keepdims=True))
        out_ref[...] = z2 - lse


def _dense_tc(asum8, xs8, deg16, bat8, w1bd, spread64, b1rep, b1, wl1, bl1,
              wl2, bl2):
    whole = lambda shape: pl.BlockSpec(shape, lambda i: tuple(0 for _ in shape))
    return pl.pallas_call(
        _dense_body,
        grid=(NT_TC,),
        in_specs=[
            pl.BlockSpec((RP, 8 * F), lambda i: (i, 0)),
            pl.BlockSpec((RP, 8 * F), lambda i: (i + NT_TC, 0)),
            pl.BlockSpec((RP, 8 * F), lambda i: (i, 0)),
            pl.BlockSpec((RP, 8 * F), lambda i: (i, 0)),
            pl.BlockSpec((RP, 8 * F), lambda i: (i + NT_TC, 0)),
            pl.BlockSpec((RP, 8), lambda i: (i, 0)),
            whole((8 * F, 8 * HID)),
            whole((8 * F, 8 * HID)),
            whole((1, 8 * HID)),
            whole((1, HID)),
            whole((HID, HID)),
            whole((1, HID)),
            whole((HID, OUT)),
            whole((1, OUT)),
        ],
        out_specs=pl.BlockSpec((G, OUT), lambda i: (0, 0)),
        out_shape=jax.ShapeDtypeStruct((G, OUT), jnp.float32),
        scratch_shapes=[
            pltpu.VMEM((G, HID), jnp.float32),
            pltpu.VMEM((G, 1), jnp.float32),
        ],
    )(asum8, asum8, xs8, deg16, deg16, bat8, w1bd, spread64, b1rep, b1,
      wl1, bl1, wl2, bl2)


def kernel(x, edge_index, batch, W1, b1, W_lin1, b_lin1, W_lin2, b_lin2):
    f32 = jnp.float32
    x8 = jnp.pad(x.reshape(N // 8, 8 * IN_DIM), ((0, NP - N // 8), (0, 0)))
    bat8 = jnp.pad(batch.reshape(N // 8, 8), ((0, NP - N // 8), (0, 0)),
                   constant_values=G)

    lanes = jnp.arange(8 * F)
    spread16 = (lanes[:, None] == (lanes[None, :] // F) * F).astype(f32)
    spread64 = (lanes[:, None]
                == (jnp.arange(8 * HID)[None, :] // HID) * F).astype(f32)
    jj = jnp.arange(8 * IN_DIM)
    perm = jnp.zeros((8 * IN_DIM, 8 * F), f32).at[
        jj, (jj // IN_DIM) * F + jj % IN_DIM].set(1.0)
    b1rep = jnp.tile(b1, 8).reshape(1, 8 * HID)

    deg_p = _deg_kernel(edge_index)
    deg16 = deg_p.reshape(2 * NP, 8 * F)
    xs8 = _prep_tc(deg16, x8, spread16, perm)
    asum_p = _agg_kernel(edge_index, xs8.reshape(N2, F),
                         jnp.zeros((N2, F), f32))

    w1bd = jnp.zeros((8 * F, 8 * HID), f32)
    for k in range(8):
        w1bd = w1bd.at[F * k:F * k + IN_DIM, HID * k:HID * (k + 1)].set(W1)
    return _dense_tc(asum_p.reshape(2 * NP, 8 * F), xs8, deg16, bat8,
                     w1bd, spread64, b1rep, b1.reshape(1, HID),
                     W_lin1, b_lin1.reshape(1, HID),
                     W_lin2, b_lin2.reshape(1, OUT))
